# Initial kernel scaffold; baseline (speedup 1.0000x reference)
import jax
import jax.numpy as jnp
from jax.experimental import pallas as pl


def kernel(x, edge_index, bn_in_g, bn_in_b, W_proj, b_proj, W1, b1, bn1_g, bn1_b, W2, b2, bn2_g, bn2_b):
    raise NotImplementedError("write your pallas kernel here")


# speedup vs baseline: 8.1355x; 8.1355x over previous
import functools

import jax
import jax.numpy as jnp
from jax import lax
from jax.experimental import pallas as pl
from jax.experimental.pallas import tpu as pltpu
from jax.experimental.pallas import tpu_sc as plsc

_NC, _NS, _L = 2, 16, 16
_NW = _NC * _NS
_CH = 128
_CPT = 80
_CBLK = 16
_EPAD = _NW * _CPT * _CH
_NACC = 10240
_DUMMY = 10016


def _mesh():
    return plsc.VectorSubcoreMesh(core_axis_name="c", subcore_axis_name="s",
                                  num_cores=_NC, num_subcores=_NS)


def _sc_degree(dst3, ones_row, zeros_col):
    n_tile = _NACC // _NS

    @functools.partial(
        pl.kernel,
        out_type=jax.ShapeDtypeStruct((_NC, _NACC), jnp.float32),
        mesh=_mesh(),
        scratch_types=[
            pltpu.VMEM((_CPT, _CH), jnp.int32),
            pltpu.VMEM((_CH,), jnp.float32),
            pltpu.VMEM((n_tile,), jnp.float32),
            pltpu.VMEM_SHARED((_NACC,), jnp.float32),
        ],
    )
    def k(dst_hbm, ones_hbm, zeros_hbm, out_hbm, idx_v, ones_v, stage_v, acc_sh):
        cid = lax.axis_index("c")
        sid = lax.axis_index("s")
        wid = cid * _NS + sid
        pltpu.sync_copy(ones_hbm, ones_v)
        pltpu.sync_copy(zeros_hbm, stage_v)
        pltpu.sync_copy(stage_v, acc_sh.at[pl.ds(sid * n_tile, n_tile)])
        pltpu.sync_copy(dst_hbm.at[wid], idx_v)
        plsc.subcore_barrier()

        def body(c, carry):
            pltpu.sync_copy(ones_v, acc_sh.at[idx_v.at[c]], add=True)
            return carry

        lax.fori_loop(0, _CPT, body, 0)
        plsc.subcore_barrier()
        pltpu.sync_copy(acc_sh.at[pl.ds(sid * n_tile, n_tile)], stage_v)
        pltpu.sync_copy(stage_v, out_hbm.at[cid, pl.ds(sid * n_tile, n_tile)])

    return k(dst3, ones_row, zeros_col)


def _sc_scatter(p, src3, dst3, zeros_blk):
    n_tile = _NACC // _NS

    @functools.partial(
        pl.kernel,
        out_type=jax.ShapeDtypeStruct((_NC, _NACC, 128), jnp.float32),
        mesh=_mesh(),
        scratch_types=[
            pltpu.VMEM((_CBLK, _CH), jnp.int32),
            pltpu.VMEM((_CBLK, _CH), jnp.int32),
            pltpu.VMEM((_CH, 128), jnp.float32),
            pltpu.VMEM((_CH, 128), jnp.float32),
            pltpu.VMEM_SHARED((_NACC, 128), jnp.float32),
            pltpu.SemaphoreType.DMA,
            pltpu.SemaphoreType.DMA,
        ],
    )
    def k(p_hbm, src_hbm, dst_hbm, zeros_hbm, out_hbm,
          src_v, dst_v, rows0, rows1, acc_sh, sem0, sem1):
        cid = lax.axis_index("c")
        sid = lax.axis_index("s")
        wid = cid * _NS + sid
        pltpu.sync_copy(zeros_hbm, rows0)
        for j in range(n_tile // _CH):
            pltpu.sync_copy(rows0, acc_sh.at[pl.ds(sid * n_tile + j * _CH, _CH)])
        plsc.subcore_barrier()

        def outer(sb, carry):
            pltpu.sync_copy(src_hbm.at[wid, pl.ds(sb * _CBLK, _CBLK)], src_v)
            pltpu.sync_copy(dst_hbm.at[wid, pl.ds(sb * _CBLK, _CBLK)], dst_v)

            def body(i, carry2):
                c0 = 2 * i
                g0 = pltpu.async_copy(p_hbm.at[src_v.at[c0]], rows0, sem0)
                g1 = pltpu.async_copy(p_hbm.at[src_v.at[c0 + 1]], rows1, sem1)
                g0.wait()
                pltpu.sync_copy(rows0, acc_sh.at[dst_v.at[c0]], add=True)
                g1.wait()
                pltpu.sync_copy(rows1, acc_sh.at[dst_v.at[c0 + 1]], add=True)
                return carry2

            lax.fori_loop(0, _CBLK // 2, body, 0)
            return carry

        lax.fori_loop(0, _CPT // _CBLK, outer, 0)
        plsc.subcore_barrier()
        for j in range(n_tile // _CH):
            pltpu.sync_copy(acc_sh.at[pl.ds(sid * n_tile + j * _CH, _CH)], rows0)
            pltpu.sync_copy(rows0, out_hbm.at[cid, pl.ds(sid * n_tile + j * _CH, _CH)])

    return k(p, src3, dst3, zeros_blk)


def _bn(h, g, b):
    m = jnp.mean(h, axis=0, keepdims=True)
    c = h - m
    v = jnp.mean(c * c, axis=0, keepdims=True)
    return c * lax.rsqrt(v + 1e-5) * g[None, :] + b[None, :]


def _matT(h, w):
    return lax.dot_general(h, w, (((1,), (1,)), ((), ())),
                           precision=lax.Precision.HIGHEST,
                           preferred_element_type=jnp.float32)


def _tc_pre(x, degp, bng, bnb, wp, bp, w1, *, interpret=False):
    n = x.shape[0]

    def body(x_ref, degp_ref, bng_ref, bnb_ref, wp_ref, bp_ref, w1_ref,
             p1_ref, dinv_ref):
        deg = degp_ref[0:1, :] + degp_ref[1:2, :] + 1.0
        dinv = lax.rsqrt(deg).reshape(_NACC, 1)[:n]
        h = _bn(x_ref[...], bng_ref[...], bnb_ref[...])
        h = jnp.maximum(_matT(h, wp_ref[...]) + bp_ref[...][None, :], 0.0)
        p1_ref[...] = dinv * _matT(h, w1_ref[...])
        dinv_ref[...] = dinv

    return pl.pallas_call(
        body,
        out_shape=[jax.ShapeDtypeStruct((n, 128), jnp.float32),
                   jax.ShapeDtypeStruct((n, 1), jnp.float32)],
        interpret=interpret,
    )(x, degp, bng, bnb, wp, bp, w1)


def _tc_mid(sp, p1, dinv, b1, bng, bnb, w2, *, interpret=False):
    n = p1.shape[0]

    def body(sp_ref, p1_ref, dinv_ref, b1_ref, bng_ref, bnb_ref, w2_ref, p2_ref):
        s = sp_ref[0, :n, :] + sp_ref[1, :n, :]
        dinv = dinv_ref[...]
        out1 = dinv * (s + p1_ref[...]) + b1_ref[...][None, :]
        h1 = jnp.maximum(_bn(out1, bng_ref[...], bnb_ref[...]), 0.0)
        p2_ref[...] = dinv * _matT(h1, w2_ref[...])

    return pl.pallas_call(
        body,
        out_shape=jax.ShapeDtypeStruct((n, 128), jnp.float32),
        interpret=interpret,
    )(sp, p1, dinv, b1, bng, bnb, w2)


def _tc_post(sp, p2, dinv, b2, bng, bnb, *, interpret=False):
    n = p2.shape[0]

    def body(sp_ref, p2_ref, dinv_ref, b2_ref, bng_ref, bnb_ref, out_ref):
        s = sp_ref[0, :n, :] + sp_ref[1, :n, :]
        out2 = dinv_ref[...] * (s + p2_ref[...]) + b2_ref[...][None, :]
        out_ref[...] = _bn(out2, bng_ref[...], bnb_ref[...])

    return pl.pallas_call(
        body,
        out_shape=jax.ShapeDtypeStruct((n, 128), jnp.float32),
        interpret=interpret,
    )(sp, p2, dinv, b2, bng, bnb)


def kernel(x, edge_index, bn_in_g, bn_in_b, W_proj, b_proj, W1, b1,
           bn1_g, bn1_b, W2, b2, bn2_g, bn2_b):
    e = edge_index.shape[1]
    pad = _EPAD - e
    src3 = jnp.concatenate(
        [edge_index[0], jnp.zeros((pad,), edge_index.dtype)]).reshape(_NW, _CPT, _CH)
    dst3 = jnp.concatenate(
        [edge_index[1], jnp.full((pad,), _DUMMY, edge_index.dtype)]).reshape(_NW, _CPT, _CH)
    ones_row = jnp.ones((_CH,), jnp.float32)
    zeros_col = jnp.zeros((_NACC // _NS,), jnp.float32)
    zeros_blk = jnp.zeros((_CH, 128), jnp.float32)

    degp = _sc_degree(dst3, ones_row, zeros_col)
    p1, dinv = _tc_pre(x, degp, bn_in_g, bn_in_b, W_proj, b_proj, W1)
    s1p = _sc_scatter(p1, src3, dst3, zeros_blk)
    p2 = _tc_mid(s1p, p1, dinv, b1, bn1_g, bn1_b, W2)
    s2p = _sc_scatter(p2, src3, dst3, zeros_blk)
    return _tc_post(s2p, p2, dinv, b2, bn2_g, bn2_b)


# speedup vs baseline: 8.3538x; 1.0268x over previous
import functools

import jax
import jax.numpy as jnp
from jax import lax
from jax.experimental import pallas as pl
from jax.experimental.pallas import tpu as pltpu
from jax.experimental.pallas import tpu_sc as plsc

_NC, _NS, _L = 2, 16, 16
_NW = _NC * _NS
_CH = 128
_CPT = 80
_CBLK = 16
_EPAD = _NW * _CPT * _CH
_NACC = 10240
_DUMMY = 10016


def _mesh():
    return plsc.VectorSubcoreMesh(core_axis_name="c", subcore_axis_name="s",
                                  num_cores=_NC, num_subcores=_NS)


def _sc_degree(dst3, ones_row, zeros_col):
    n_tile = _NACC // _NS

    @functools.partial(
        pl.kernel,
        out_type=jax.ShapeDtypeStruct((_NC, _NACC), jnp.float32),
        mesh=_mesh(),
        scratch_types=[
            pltpu.VMEM((_CPT, _CH), jnp.int32),
            pltpu.VMEM((_CH,), jnp.float32),
            pltpu.VMEM((n_tile,), jnp.float32),
            pltpu.VMEM_SHARED((_NACC,), jnp.float32),
        ],
    )
    def k(dst_hbm, ones_hbm, zeros_hbm, out_hbm, idx_v, ones_v, stage_v, acc_sh):
        cid = lax.axis_index("c")
        sid = lax.axis_index("s")
        wid = cid * _NS + sid
        pltpu.sync_copy(ones_hbm, ones_v)
        pltpu.sync_copy(zeros_hbm, stage_v)
        pltpu.sync_copy(stage_v, acc_sh.at[pl.ds(sid * n_tile, n_tile)])
        pltpu.sync_copy(dst_hbm.at[wid], idx_v)
        plsc.subcore_barrier()

        def body(c, carry):
            pltpu.sync_copy(ones_v, acc_sh.at[idx_v.at[c]], add=True)
            return carry

        lax.fori_loop(0, _CPT, body, 0)
        plsc.subcore_barrier()
        pltpu.sync_copy(acc_sh.at[pl.ds(sid * n_tile, n_tile)], stage_v)
        pltpu.sync_copy(stage_v, out_hbm.at[cid, pl.ds(sid * n_tile, n_tile)])

    return k(dst3, ones_row, zeros_col)


def _sc_scatter(p, src3, dst3, zeros_blk):
    n_tile = _NACC // _NS

    @functools.partial(
        pl.kernel,
        out_type=jax.ShapeDtypeStruct((_NC, _NACC, 128), jnp.float32),
        mesh=_mesh(),
        scratch_types=[
            pltpu.VMEM((_CBLK, _CH), jnp.int32),
            pltpu.VMEM((_CBLK, _CH), jnp.int32),
            pltpu.VMEM((_CH, 128), jnp.float32),
            pltpu.VMEM((_CH, 128), jnp.float32),
            pltpu.VMEM_SHARED((_NACC, 128), jnp.float32),
            pltpu.SemaphoreType.DMA,
            pltpu.SemaphoreType.DMA,
            pltpu.SemaphoreType.DMA,
            pltpu.SemaphoreType.DMA,
        ],
    )
    def k(p_hbm, src_hbm, dst_hbm, zeros_hbm, out_hbm,
          src_v, dst_v, rows0, rows1, acc_sh, sem0, sem1, sem2, sem3):
        cid = lax.axis_index("c")
        sid = lax.axis_index("s")
        wid = cid * _NS + sid
        pltpu.sync_copy(zeros_hbm, rows0)
        for j in range(n_tile // _CH):
            pltpu.sync_copy(rows0, acc_sh.at[pl.ds(sid * n_tile + j * _CH, _CH)])
        plsc.subcore_barrier()

        def outer(sb, carry):
            pltpu.sync_copy(src_hbm.at[wid, pl.ds(sb * _CBLK, _CBLK)], src_v)
            pltpu.sync_copy(dst_hbm.at[wid, pl.ds(sb * _CBLK, _CBLK)], dst_v)

            def body(i, carry2):
                c0 = 2 * i
                g0 = pltpu.async_copy(p_hbm.at[src_v.at[c0]], rows0, sem0)
                g1 = pltpu.async_copy(p_hbm.at[src_v.at[c0 + 1]], rows1, sem1)
                g0.wait()
                s0 = pltpu.async_copy(rows0, acc_sh.at[dst_v.at[c0]], sem2,
                                      add=True)
                g1.wait()
                s1 = pltpu.async_copy(rows1, acc_sh.at[dst_v.at[c0 + 1]], sem3,
                                      add=True)
                s0.wait()
                s1.wait()
                return carry2

            lax.fori_loop(0, _CBLK // 2, body, 0)
            return carry

        lax.fori_loop(0, _CPT // _CBLK, outer, 0)
        plsc.subcore_barrier()
        for j in range(n_tile // _CH):
            pltpu.sync_copy(acc_sh.at[pl.ds(sid * n_tile + j * _CH, _CH)], rows0)
            pltpu.sync_copy(rows0, out_hbm.at[cid, pl.ds(sid * n_tile + j * _CH, _CH)])

    return k(p, src3, dst3, zeros_blk)


def _bn(h, g, b):
    m = jnp.mean(h, axis=0, keepdims=True)
    c = h - m
    v = jnp.mean(c * c, axis=0, keepdims=True)
    return c / jnp.sqrt(v + 1e-5) * g[None, :] + b[None, :]


def _matT(h, w):
    return lax.dot_general(h, w, (((1,), (1,)), ((), ())),
                           precision=lax.Precision.HIGHEST,
                           preferred_element_type=jnp.float32)


def _tc_pre(x, degp, bng, bnb, wp, bp, w1, *, interpret=False):
    n = x.shape[0]

    def body(x_ref, degp_ref, bng_ref, bnb_ref, wp_ref, bp_ref, w1_ref,
             p1_ref, dinv_ref):
        deg = degp_ref[0:1, :] + degp_ref[1:2, :] + 1.0
        dinv = (1.0 / jnp.sqrt(deg)).reshape(_NACC, 1)[:n]
        h = _bn(x_ref[...], bng_ref[...], bnb_ref[...])
        h = jnp.maximum(_matT(h, wp_ref[...]) + bp_ref[...][None, :], 0.0)
        p1_ref[...] = dinv * _matT(h, w1_ref[...])
        dinv_ref[...] = dinv

    return pl.pallas_call(
        body,
        out_shape=[jax.ShapeDtypeStruct((n, 128), jnp.float32),
                   jax.ShapeDtypeStruct((n, 1), jnp.float32)],
        interpret=interpret,
    )(x, degp, bng, bnb, wp, bp, w1)


def _tc_mid(sp, p1, dinv, b1, bng, bnb, w2, *, interpret=False):
    n = p1.shape[0]

    def body(sp_ref, p1_ref, dinv_ref, b1_ref, bng_ref, bnb_ref, w2_ref, p2_ref):
        s = sp_ref[0, :n, :] + sp_ref[1, :n, :]
        dinv = dinv_ref[...]
        out1 = dinv * (s + p1_ref[...]) + b1_ref[...][None, :]
        h1 = jnp.maximum(_bn(out1, bng_ref[...], bnb_ref[...]), 0.0)
        p2_ref[...] = dinv * _matT(h1, w2_ref[...])

    return pl.pallas_call(
        body,
        out_shape=jax.ShapeDtypeStruct((n, 128), jnp.float32),
        interpret=interpret,
    )(sp, p1, dinv, b1, bng, bnb, w2)


def _tc_post(sp, p2, dinv, b2, bng, bnb, *, interpret=False):
    n = p2.shape[0]

    def body(sp_ref, p2_ref, dinv_ref, b2_ref, bng_ref, bnb_ref, out_ref):
        s = sp_ref[0, :n, :] + sp_ref[1, :n, :]
        out2 = dinv_ref[...] * (s + p2_ref[...]) + b2_ref[...][None, :]
        out_ref[...] = _bn(out2, bng_ref[...], bnb_ref[...])

    return pl.pallas_call(
        body,
        out_shape=jax.ShapeDtypeStruct((n, 128), jnp.float32),
        interpret=interpret,
    )(sp, p2, dinv, b2, bng, bnb)


def kernel(x, edge_index, bn_in_g, bn_in_b, W_proj, b_proj, W1, b1,
           bn1_g, bn1_b, W2, b2, bn2_g, bn2_b):
    e = edge_index.shape[1]
    pad = _EPAD - e
    src3 = jnp.concatenate(
        [edge_index[0], jnp.zeros((pad,), edge_index.dtype)]).reshape(_NW, _CPT, _CH)
    dst3 = jnp.concatenate(
        [edge_index[1], jnp.full((pad,), _DUMMY, edge_index.dtype)]).reshape(_NW, _CPT, _CH)
    ones_row = jnp.ones((_CH,), jnp.float32)
    zeros_col = jnp.zeros((_NACC // _NS,), jnp.float32)
    zeros_blk = jnp.zeros((_CH, 128), jnp.float32)

    degp = _sc_degree(dst3, ones_row, zeros_col)
    p1, dinv = _tc_pre(x, degp, bn_in_g, bn_in_b, W_proj, b_proj, W1)
    s1p = _sc_scatter(p1, src3, dst3, zeros_blk)
    p2 = _tc_mid(s1p, p1, dinv, b1, bn1_g, bn1_b, W2)
    s2p = _sc_scatter(p2, src3, dst3, zeros_blk)
    return _tc_post(s2p, p2, dinv, b2, bn2_g, bn2_b)


# speedup vs baseline: 10.5274x; 1.2602x over previous
import functools

import jax
import jax.numpy as jnp
from jax import lax
from jax.experimental import pallas as pl
from jax.experimental.pallas import tpu as pltpu
from jax.experimental.pallas import tpu_sc as plsc

_NC, _NS, _L = 2, 16, 16
_NW = _NC * _NS
_CH = 128
_CPT = 80
_CPS = 160
_GRP = 8
_EPAD = _NW * _CPT * _CH
_NACC = 10240
_DUMMY = 10016


def _mesh():
    return plsc.VectorSubcoreMesh(core_axis_name="c", subcore_axis_name="s",
                                  num_cores=_NC, num_subcores=_NS)


def _sc_degree(dst3, ones_row, zeros_col):
    n_tile = _NACC // _NS

    @functools.partial(
        pl.kernel,
        out_type=jax.ShapeDtypeStruct((_NC, _NACC), jnp.float32),
        mesh=_mesh(),
        scratch_types=[
            pltpu.VMEM((_CPT, _CH), jnp.int32),
            pltpu.VMEM((_CH,), jnp.float32),
            pltpu.VMEM((n_tile,), jnp.float32),
            pltpu.VMEM_SHARED((_NACC,), jnp.float32),
        ],
    )
    def k(dst_hbm, ones_hbm, zeros_hbm, out_hbm, idx_v, ones_v, stage_v, acc_sh):
        cid = lax.axis_index("c")
        sid = lax.axis_index("s")
        wid = cid * _NS + sid
        pltpu.sync_copy(ones_hbm, ones_v)
        pltpu.sync_copy(zeros_hbm, stage_v)
        pltpu.sync_copy(stage_v, acc_sh.at[pl.ds(sid * n_tile, n_tile)])
        pltpu.sync_copy(dst_hbm.at[wid], idx_v)
        plsc.subcore_barrier()

        def body(c, carry):
            pltpu.sync_copy(ones_v, acc_sh.at[idx_v.at[c]], add=True)
            return carry

        lax.fori_loop(0, _CPT, body, 0)
        plsc.subcore_barrier()
        pltpu.sync_copy(acc_sh.at[pl.ds(sid * n_tile, n_tile)], stage_v)
        pltpu.sync_copy(stage_v, out_hbm.at[cid, pl.ds(sid * n_tile, n_tile)])

    return k(dst3, ones_row, zeros_col)


def _sc_scatter(pcat, srclo3, srchi3, dst3s, zeros_blk):
    n_tile = _NACC // _NS

    @functools.partial(
        pl.kernel,
        out_type=jax.ShapeDtypeStruct((_NC, _NACC, 64), jnp.float32),
        mesh=_mesh(),
        compiler_params=pltpu.CompilerParams(use_tc_tiling_on_sc=False),
        scratch_types=[
            pltpu.VMEM((_GRP, _CH), jnp.int32),
            pltpu.VMEM((_GRP, _CH), jnp.int32),
            [pltpu.VMEM((_CH, 64), jnp.float32) for _ in range(_GRP)],
            pltpu.VMEM_SHARED((_NACC, 64), jnp.float32),
            pltpu.SemaphoreType.DMA,
            pltpu.SemaphoreType.DMA,
        ],
    )
    def k(p_hbm, srclo_hbm, srchi_hbm, dst_hbm, zeros_hbm, out_hbm,
          src_v, dst_v, bufs, acc_sh, gsem, ssem):
        cid = lax.axis_index("c")
        sid = lax.axis_index("s")
        pltpu.sync_copy(zeros_hbm, bufs[0])
        for j in range(n_tile // _CH):
            pltpu.sync_copy(bufs[0],
                            acc_sh.at[pl.ds(sid * n_tile + j * _CH, _CH)])
        plsc.subcore_barrier()

        def group(g, carry):
            @pl.when(cid == 0)
            def _():
                pltpu.sync_copy(srclo_hbm.at[sid, pl.ds(g * _GRP, _GRP)], src_v)

            @pl.when(cid == 1)
            def _():
                pltpu.sync_copy(srchi_hbm.at[sid, pl.ds(g * _GRP, _GRP)], src_v)

            pltpu.sync_copy(dst_hbm.at[sid, pl.ds(g * _GRP, _GRP)], dst_v)
            gathers = [pltpu.async_copy(p_hbm.at[src_v.at[kk]], bufs[kk], gsem)
                       for kk in range(_GRP)]
            for gth in gathers:
                gth.wait()
            scats = [pltpu.async_copy(bufs[kk], acc_sh.at[dst_v.at[kk]], ssem,
                                      add=True)
                     for kk in range(_GRP)]
            for s in scats:
                s.wait()
            return carry

        lax.fori_loop(0, _CPS // _GRP, group, 0)
        plsc.subcore_barrier()
        for j in range(n_tile // _CH):
            pltpu.sync_copy(acc_sh.at[pl.ds(sid * n_tile + j * _CH, _CH)],
                            bufs[0])
            pltpu.sync_copy(bufs[0],
                            out_hbm.at[cid, pl.ds(sid * n_tile + j * _CH, _CH)])

    return k(pcat, srclo3, srchi3, dst3s, zeros_blk)


def _bn(h, g, b):
    m = jnp.mean(h, axis=0, keepdims=True)
    c = h - m
    v = jnp.mean(c * c, axis=0, keepdims=True)
    return c / jnp.sqrt(v + 1e-5) * g[None, :] + b[None, :]


def _matT(h, w):
    return lax.dot_general(h, w, (((1,), (1,)), ((), ())),
                           precision=lax.Precision.HIGHEST,
                           preferred_element_type=jnp.float32)


def _tc_pre(x, degp, bng, bnb, wp, bp, w1, *, interpret=False):
    n = x.shape[0]

    def body(x_ref, degp_ref, bng_ref, bnb_ref, wp_ref, bp_ref, w1_ref,
             p1_ref, dinv_ref):
        deg = degp_ref[0:1, :] + degp_ref[1:2, :] + 1.0
        dinv = (1.0 / jnp.sqrt(deg)).reshape(_NACC, 1)[:n]
        h = _bn(x_ref[...], bng_ref[...], bnb_ref[...])
        h = jnp.maximum(_matT(h, wp_ref[...]) + bp_ref[...][None, :], 0.0)
        p1 = dinv * _matT(h, w1_ref[...])
        p1_ref[0:n, :] = p1[:, 0:64]
        p1_ref[n:2 * n, :] = p1[:, 64:128]
        dinv_ref[...] = dinv

    return pl.pallas_call(
        body,
        out_shape=[jax.ShapeDtypeStruct((2 * n, 64), jnp.float32),
                   jax.ShapeDtypeStruct((n, 1), jnp.float32)],
        compiler_params=pltpu.CompilerParams(vmem_limit_bytes=100 * 1024 * 1024),
        interpret=interpret,
    )(x, degp, bng, bnb, wp, bp, w1)


def _tc_mid(sp, p1, dinv, b1, bng, bnb, w2, *, interpret=False):
    n = dinv.shape[0]

    def body(sp_ref, p1_ref, dinv_ref, b1_ref, bng_ref, bnb_ref, w2_ref, p2_ref):
        s = jnp.concatenate([sp_ref[0, :n, :], sp_ref[1, :n, :]], axis=1)
        p1 = jnp.concatenate([p1_ref[0:n, :], p1_ref[n:2 * n, :]], axis=1)
        dinv = dinv_ref[...]
        out1 = dinv * (s + p1) + b1_ref[...][None, :]
        h1 = jnp.maximum(_bn(out1, bng_ref[...], bnb_ref[...]), 0.0)
        p2 = dinv * _matT(h1, w2_ref[...])
        p2_ref[0:n, :] = p2[:, 0:64]
        p2_ref[n:2 * n, :] = p2[:, 64:128]

    return pl.pallas_call(
        body,
        out_shape=jax.ShapeDtypeStruct((2 * n, 64), jnp.float32),
        compiler_params=pltpu.CompilerParams(vmem_limit_bytes=100 * 1024 * 1024),
        interpret=interpret,
    )(sp, p1, dinv, b1, bng, bnb, w2)


def _tc_post(sp, p2, dinv, b2, bng, bnb, *, interpret=False):
    n = dinv.shape[0]

    def body(sp_ref, p2_ref, dinv_ref, b2_ref, bng_ref, bnb_ref, out_ref):
        s = jnp.concatenate([sp_ref[0, :n, :], sp_ref[1, :n, :]], axis=1)
        p2 = jnp.concatenate([p2_ref[0:n, :], p2_ref[n:2 * n, :]], axis=1)
        out2 = dinv_ref[...] * (s + p2) + b2_ref[...][None, :]
        out_ref[...] = _bn(out2, bng_ref[...], bnb_ref[...])

    return pl.pallas_call(
        body,
        out_shape=jax.ShapeDtypeStruct((n, 128), jnp.float32),
        compiler_params=pltpu.CompilerParams(vmem_limit_bytes=100 * 1024 * 1024),
        interpret=interpret,
    )(sp, p2, dinv, b2, bng, bnb)


def kernel(x, edge_index, bn_in_g, bn_in_b, W_proj, b_proj, W1, b1,
           bn1_g, bn1_b, W2, b2, bn2_g, bn2_b):
    n = x.shape[0]
    e = edge_index.shape[1]
    pad = _EPAD - e
    src_p = jnp.concatenate([edge_index[0], jnp.zeros((pad,), edge_index.dtype)])
    dst_p = jnp.concatenate([edge_index[1], jnp.full((pad,), _DUMMY, edge_index.dtype)])
    src3 = src_p.reshape(_NW, _CPT, _CH)
    dst3 = dst_p.reshape(_NW, _CPT, _CH)
    srclo3 = src_p.reshape(_NS, _CPS, _CH)
    srchi3 = srclo3 + n
    dst3s = dst_p.reshape(_NS, _CPS, _CH)
    ones_row = jnp.ones((_CH,), jnp.float32)
    zeros_col = jnp.zeros((_NACC // _NS,), jnp.float32)
    zeros_blk = jnp.zeros((_CH, 64), jnp.float32)

    degp = _sc_degree(dst3, ones_row, zeros_col)
    p1, dinv = _tc_pre(x, degp, bn_in_g, bn_in_b, W_proj, b_proj, W1)
    s1p = _sc_scatter(p1, srclo3, srchi3, dst3s, zeros_blk)
    p2 = _tc_mid(s1p, p1, dinv, b1, bn1_g, bn1_b, W2)
    s2p = _sc_scatter(p2, srclo3, srchi3, dst3s, zeros_blk)
    return _tc_post(s2p, p2, dinv, b2, bn2_g, bn2_b)


# speedup vs baseline: 11.5469x; 1.0968x over previous
import functools

import jax
import jax.numpy as jnp
from jax import lax
from jax.experimental import pallas as pl
from jax.experimental.pallas import tpu as pltpu
from jax.experimental.pallas import tpu_sc as plsc

_NC, _NS, _L = 2, 16, 16
_NW = _NC * _NS
_CH = 128
_CPT = 80
_CPS = 160
_BNK = 4
_EPAD = _NW * _CPT * _CH
_NACC = 10240
_DUMMY = 10016


def _mesh():
    return plsc.VectorSubcoreMesh(core_axis_name="c", subcore_axis_name="s",
                                  num_cores=_NC, num_subcores=_NS)


def _sc_degree(dst3, ones_row, zeros_col):
    n_tile = _NACC // _NS

    @functools.partial(
        pl.kernel,
        out_type=jax.ShapeDtypeStruct((_NC, _NACC), jnp.float32),
        mesh=_mesh(),
        scratch_types=[
            pltpu.VMEM((_CPT, _CH), jnp.int32),
            pltpu.VMEM((_CH,), jnp.float32),
            pltpu.VMEM((n_tile,), jnp.float32),
            pltpu.VMEM_SHARED((_NACC,), jnp.float32),
        ],
    )
    def k(dst_hbm, ones_hbm, zeros_hbm, out_hbm, idx_v, ones_v, stage_v, acc_sh):
        cid = lax.axis_index("c")
        sid = lax.axis_index("s")
        wid = cid * _NS + sid
        pltpu.sync_copy(ones_hbm, ones_v)
        pltpu.sync_copy(zeros_hbm, stage_v)
        pltpu.sync_copy(stage_v, acc_sh.at[pl.ds(sid * n_tile, n_tile)])
        pltpu.sync_copy(dst_hbm.at[wid], idx_v)
        plsc.subcore_barrier()

        def body(c, carry):
            pltpu.sync_copy(ones_v, acc_sh.at[idx_v.at[c]], add=True)
            return carry

        lax.fori_loop(0, _CPT, body, 0)
        plsc.subcore_barrier()
        pltpu.sync_copy(acc_sh.at[pl.ds(sid * n_tile, n_tile)], stage_v)
        pltpu.sync_copy(stage_v, out_hbm.at[cid, pl.ds(sid * n_tile, n_tile)])

    return k(dst3, ones_row, zeros_col)


def _sc_scatter(pcat, srclo3, srchi3, dst3s, zeros_blk):
    n_tile = _NACC // _NS
    n_grp = _CPS // (2 * _BNK)

    @functools.partial(
        pl.kernel,
        out_type=jax.ShapeDtypeStruct((_NC, _NACC, 64), jnp.float32),
        mesh=_mesh(),
        compiler_params=pltpu.CompilerParams(use_tc_tiling_on_sc=False),
        scratch_types=[
            [pltpu.VMEM((_BNK, _CH), jnp.int32) for _ in range(4)],
            [pltpu.VMEM((_CH, 64), jnp.float32) for _ in range(2 * _BNK)],
            pltpu.VMEM_SHARED((_NACC, 64), jnp.float32),
            [pltpu.SemaphoreType.DMA for _ in range(4)],
        ],
    )
    def k(p_hbm, srclo_hbm, srchi_hbm, dst_hbm, zeros_hbm, out_hbm,
          idx_v, bufs, acc_sh, sems):
        cid = lax.axis_index("c")
        sid = lax.axis_index("s")
        src_a, dst_a, src_b, dst_b = idx_v
        gsem_a, ssem_a, gsem_b, ssem_b = sems
        bufs_a, bufs_b = bufs[:_BNK], bufs[_BNK:]
        pltpu.sync_copy(zeros_hbm, bufs[0])
        for j in range(n_tile // _CH):
            pltpu.sync_copy(bufs[0],
                            acc_sh.at[pl.ds(sid * n_tile + j * _CH, _CH)])
        plsc.subcore_barrier()

        def stage(g, off, src_v, dst_v):
            @pl.when(cid == 0)
            def _():
                pltpu.sync_copy(
                    srclo_hbm.at[sid, pl.ds(g * 2 * _BNK + off, _BNK)], src_v)

            @pl.when(cid == 1)
            def _():
                pltpu.sync_copy(
                    srchi_hbm.at[sid, pl.ds(g * 2 * _BNK + off, _BNK)], src_v)

            pltpu.sync_copy(dst_hbm.at[sid, pl.ds(g * 2 * _BNK + off, _BNK)],
                            dst_v)

        def fire_g(src_v, bank, sem):
            return [pltpu.async_copy(p_hbm.at[src_v.at[kk]], bank[kk], sem)
                    for kk in range(_BNK)]

        def fire_s(dst_v, bank, sem):
            return [pltpu.async_copy(bank[kk], acc_sh.at[dst_v.at[kk]], sem,
                                     add=True)
                    for kk in range(_BNK)]

        def wait_g(src_v, bank, sem):
            for kk in range(_BNK):
                pltpu.make_async_copy(p_hbm.at[src_v.at[kk]], bank[kk],
                                      sem).wait()

        def drain(ds):
            for d in ds:
                d.wait()

        stage(0, 0, src_a, dst_a)
        stage(0, _BNK, src_b, dst_b)
        fire_g(src_a, bufs_a, gsem_a)

        def group(g, carry):
            gb = fire_g(src_b, bufs_b, gsem_b)
            wait_g(src_a, bufs_a, gsem_a)
            drain(fire_s(dst_a, bufs_a, ssem_a))
            stage(g + 1, 0, src_a, dst_a)
            fire_g(src_a, bufs_a, gsem_a)
            drain(gb)
            drain(fire_s(dst_b, bufs_b, ssem_b))
            stage(g + 1, _BNK, src_b, dst_b)
            return carry

        lax.fori_loop(0, n_grp - 1, group, 0)
        gb = fire_g(src_b, bufs_b, gsem_b)
        wait_g(src_a, bufs_a, gsem_a)
        drain(fire_s(dst_a, bufs_a, ssem_a))
        drain(gb)
        drain(fire_s(dst_b, bufs_b, ssem_b))
        plsc.subcore_barrier()
        for j in range(n_tile // _CH):
            pltpu.sync_copy(acc_sh.at[pl.ds(sid * n_tile + j * _CH, _CH)],
                            bufs[0])
            pltpu.sync_copy(bufs[0],
                            out_hbm.at[cid, pl.ds(sid * n_tile + j * _CH, _CH)])

    return k(pcat, srclo3, srchi3, dst3s, zeros_blk)


def _bn(h, g, b):
    m = jnp.mean(h, axis=0, keepdims=True)
    c = h - m
    v = jnp.mean(c * c, axis=0, keepdims=True)
    return c / jnp.sqrt(v + 1e-5) * g[None, :] + b[None, :]


def _matT(h, w):
    return lax.dot_general(h, w, (((1,), (1,)), ((), ())),
                           precision=lax.Precision.HIGHEST,
                           preferred_element_type=jnp.float32)


def _tc_pre(x, degp, bng, bnb, wp, bp, w1, *, interpret=False):
    n = x.shape[0]

    def body(x_ref, degp_ref, bng_ref, bnb_ref, wp_ref, bp_ref, w1_ref,
             p1_ref, dinv_ref):
        deg = degp_ref[0:1, :] + degp_ref[1:2, :] + 1.0
        dinv = (1.0 / jnp.sqrt(deg)).reshape(_NACC, 1)[:n]
        h = _bn(x_ref[...], bng_ref[...], bnb_ref[...])
        h = jnp.maximum(_matT(h, wp_ref[...]) + bp_ref[...][None, :], 0.0)
        p1 = dinv * _matT(h, w1_ref[...])
        p1_ref[0:n, :] = p1[:, 0:64]
        p1_ref[n:2 * n, :] = p1[:, 64:128]
        dinv_ref[...] = dinv

    return pl.pallas_call(
        body,
        out_shape=[jax.ShapeDtypeStruct((2 * n, 64), jnp.float32),
                   jax.ShapeDtypeStruct((n, 1), jnp.float32)],
        compiler_params=pltpu.CompilerParams(vmem_limit_bytes=100 * 1024 * 1024),
        interpret=interpret,
    )(x, degp, bng, bnb, wp, bp, w1)


def _tc_mid(sp, p1, dinv, b1, bng, bnb, w2, *, interpret=False):
    n = dinv.shape[0]

    def body(sp_ref, p1_ref, dinv_ref, b1_ref, bng_ref, bnb_ref, w2_ref, p2_ref):
        s = jnp.concatenate([sp_ref[0, :n, :], sp_ref[1, :n, :]], axis=1)
        p1 = jnp.concatenate([p1_ref[0:n, :], p1_ref[n:2 * n, :]], axis=1)
        dinv = dinv_ref[...]
        out1 = dinv * (s + p1) + b1_ref[...][None, :]
        h1 = jnp.maximum(_bn(out1, bng_ref[...], bnb_ref[...]), 0.0)
        p2 = dinv * _matT(h1, w2_ref[...])
        p2_ref[0:n, :] = p2[:, 0:64]
        p2_ref[n:2 * n, :] = p2[:, 64:128]

    return pl.pallas_call(
        body,
        out_shape=jax.ShapeDtypeStruct((2 * n, 64), jnp.float32),
        compiler_params=pltpu.CompilerParams(vmem_limit_bytes=100 * 1024 * 1024),
        interpret=interpret,
    )(sp, p1, dinv, b1, bng, bnb, w2)


def _tc_post(sp, p2, dinv, b2, bng, bnb, *, interpret=False):
    n = dinv.shape[0]

    def body(sp_ref, p2_ref, dinv_ref, b2_ref, bng_ref, bnb_ref, out_ref):
        s = jnp.concatenate([sp_ref[0, :n, :], sp_ref[1, :n, :]], axis=1)
        p2 = jnp.concatenate([p2_ref[0:n, :], p2_ref[n:2 * n, :]], axis=1)
        out2 = dinv_ref[...] * (s + p2) + b2_ref[...][None, :]
        out_ref[...] = _bn(out2, bng_ref[...], bnb_ref[...])

    return pl.pallas_call(
        body,
        out_shape=jax.ShapeDtypeStruct((n, 128), jnp.float32),
        compiler_params=pltpu.CompilerParams(vmem_limit_bytes=100 * 1024 * 1024),
        interpret=interpret,
    )(sp, p2, dinv, b2, bng, bnb)


def kernel(x, edge_index, bn_in_g, bn_in_b, W_proj, b_proj, W1, b1,
           bn1_g, bn1_b, W2, b2, bn2_g, bn2_b):
    n = x.shape[0]
    e = edge_index.shape[1]
    pad = _EPAD - e
    src_p = jnp.concatenate([edge_index[0], jnp.zeros((pad,), edge_index.dtype)])
    dst_p = jnp.concatenate([edge_index[1], jnp.full((pad,), _DUMMY, edge_index.dtype)])
    src3 = src_p.reshape(_NW, _CPT, _CH)
    dst3 = dst_p.reshape(_NW, _CPT, _CH)
    srclo3 = src_p.reshape(_NS, _CPS, _CH)
    srchi3 = srclo3 + n
    dst3s = dst_p.reshape(_NS, _CPS, _CH)
    ones_row = jnp.ones((_CH,), jnp.float32)
    zeros_col = jnp.zeros((_NACC // _NS,), jnp.float32)
    zeros_blk = jnp.zeros((_CH, 64), jnp.float32)

    degp = _sc_degree(dst3, ones_row, zeros_col)
    p1, dinv = _tc_pre(x, degp, bn_in_g, bn_in_b, W_proj, b_proj, W1)
    s1p = _sc_scatter(p1, srclo3, srchi3, dst3s, zeros_blk)
    p2 = _tc_mid(s1p, p1, dinv, b1, bn1_g, bn1_b, W2)
    s2p = _sc_scatter(p2, srclo3, srchi3, dst3s, zeros_blk)
    return _tc_post(s2p, p2, dinv, b2, bn2_g, bn2_b)


# speedup vs baseline: 16.2859x; 1.4104x over previous
import functools

import jax
import jax.numpy as jnp
from jax import lax
from jax.experimental import pallas as pl
from jax.experimental.pallas import tpu as pltpu
from jax.experimental.pallas import tpu_sc as plsc

_NC, _NS, _L = 2, 16, 16
_NW = _NC * _NS
_CH = 128
_CPT = 80
_CPS = 160
_BNK = 2
_EPAD = _NW * _CPT * _CH
_NACC = 10240
_DUMMY = 10016


def _mesh():
    return plsc.VectorSubcoreMesh(core_axis_name="c", subcore_axis_name="s",
                                  num_cores=_NC, num_subcores=_NS)


def _sc_degree(dst3, ones_row, zeros_col):
    n_tile = _NACC // _NS

    @functools.partial(
        pl.kernel,
        out_type=jax.ShapeDtypeStruct((_NC, _NACC), jnp.float32),
        mesh=_mesh(),
        scratch_types=[
            pltpu.VMEM((_CPT, _CH), jnp.int32),
            pltpu.VMEM((_CH,), jnp.float32),
            pltpu.VMEM((n_tile,), jnp.float32),
            pltpu.VMEM_SHARED((_NACC,), jnp.float32),
        ],
    )
    def k(dst_hbm, ones_hbm, zeros_hbm, out_hbm, idx_v, ones_v, stage_v, acc_sh):
        cid = lax.axis_index("c")
        sid = lax.axis_index("s")
        wid = cid * _NS + sid
        pltpu.sync_copy(ones_hbm, ones_v)
        pltpu.sync_copy(zeros_hbm, stage_v)
        pltpu.sync_copy(stage_v, acc_sh.at[pl.ds(sid * n_tile, n_tile)])
        pltpu.sync_copy(dst_hbm.at[wid], idx_v)
        plsc.subcore_barrier()

        def body(c, carry):
            pltpu.sync_copy(ones_v, acc_sh.at[idx_v.at[c]], add=True)
            return carry

        lax.fori_loop(0, _CPT, body, 0)
        plsc.subcore_barrier()
        pltpu.sync_copy(acc_sh.at[pl.ds(sid * n_tile, n_tile)], stage_v)
        pltpu.sync_copy(stage_v, out_hbm.at[cid, pl.ds(sid * n_tile, n_tile)])

    return k(dst3, ones_row, zeros_col)


def _sc_scatter(pstk, src3s, dst3s, zeros_blk):
    n_tile = _NACC // _NS
    n_grp = _CPS // (2 * _BNK)

    @functools.partial(
        pl.kernel,
        out_type=jax.ShapeDtypeStruct((_NC, _NACC, 64), jnp.float32),
        mesh=_mesh(),
        compiler_params=pltpu.CompilerParams(use_tc_tiling_on_sc=False),
        scratch_types=[
            [pltpu.VMEM((_BNK, _CH), jnp.int32) for _ in range(4)],
            [pltpu.VMEM((_CH, 64), jnp.float32) for _ in range(2 * _BNK)],
            pltpu.VMEM_SHARED((_NACC, 64), jnp.float32),
            pltpu.VMEM_SHARED((_NACC, 64), jnp.float32),
            [pltpu.SemaphoreType.DMA for _ in range(4)],
        ],
    )
    def k(p_hbm, src_hbm, dst_hbm, zeros_hbm, out_hbm,
          idx_v, bufs, acc_sh, p_sh, sems):
        cid = lax.axis_index("c")
        sid = lax.axis_index("s")
        src_a, dst_a, src_b, dst_b = idx_v
        gsem_a, ssem_a, gsem_b, ssem_b = sems
        bufs_a, bufs_b = bufs[:_BNK], bufs[_BNK:]
        pltpu.sync_copy(zeros_hbm, bufs[0])
        for j in range(n_tile // _CH):
            pltpu.sync_copy(bufs[0],
                            acc_sh.at[pl.ds(sid * n_tile + j * _CH, _CH)])
        pltpu.sync_copy(p_hbm.at[cid, pl.ds(sid * n_tile, n_tile)],
                        p_sh.at[pl.ds(sid * n_tile, n_tile)])
        plsc.subcore_barrier()

        def stage(g, off, src_v, dst_v):
            pltpu.sync_copy(src_hbm.at[sid, pl.ds(g * 2 * _BNK + off, _BNK)],
                            src_v)
            pltpu.sync_copy(dst_hbm.at[sid, pl.ds(g * 2 * _BNK + off, _BNK)],
                            dst_v)

        def fire_g(src_v, bank, sem):
            return [pltpu.async_copy(p_sh.at[src_v.at[kk]], bank[kk], sem)
                    for kk in range(_BNK)]

        def fire_s(dst_v, bank, sem):
            return [pltpu.async_copy(bank[kk], acc_sh.at[dst_v.at[kk]], sem,
                                     add=True)
                    for kk in range(_BNK)]

        def wait_g(src_v, bank, sem):
            for kk in range(_BNK):
                pltpu.make_async_copy(p_sh.at[src_v.at[kk]], bank[kk],
                                      sem).wait()

        def drain(ds):
            for d in ds:
                d.wait()

        stage(0, 0, src_a, dst_a)
        stage(0, _BNK, src_b, dst_b)
        fire_g(src_a, bufs_a, gsem_a)

        def group(g, carry):
            gb = fire_g(src_b, bufs_b, gsem_b)
            wait_g(src_a, bufs_a, gsem_a)
            drain(fire_s(dst_a, bufs_a, ssem_a))
            stage(g + 1, 0, src_a, dst_a)
            fire_g(src_a, bufs_a, gsem_a)
            drain(gb)
            drain(fire_s(dst_b, bufs_b, ssem_b))
            stage(g + 1, _BNK, src_b, dst_b)
            return carry

        lax.fori_loop(0, n_grp - 1, group, 0)
        gb = fire_g(src_b, bufs_b, gsem_b)
        wait_g(src_a, bufs_a, gsem_a)
        drain(fire_s(dst_a, bufs_a, ssem_a))
        drain(gb)
        drain(fire_s(dst_b, bufs_b, ssem_b))
        plsc.subcore_barrier()
        for j in range(n_tile // _CH):
            pltpu.sync_copy(acc_sh.at[pl.ds(sid * n_tile + j * _CH, _CH)],
                            bufs[0])
            pltpu.sync_copy(bufs[0],
                            out_hbm.at[cid, pl.ds(sid * n_tile + j * _CH, _CH)])

    return k(pstk, src3s, dst3s, zeros_blk)


def _bn(h, g, b):
    m = jnp.mean(h, axis=0, keepdims=True)
    c = h - m
    v = jnp.mean(c * c, axis=0, keepdims=True)
    return c / jnp.sqrt(v + 1e-5) * g[None, :] + b[None, :]


def _matT(h, w):
    return lax.dot_general(h, w, (((1,), (1,)), ((), ())),
                           precision=lax.Precision.HIGHEST,
                           preferred_element_type=jnp.float32)


def _tc_pre(x, degp, bng, bnb, wp, bp, w1, *, interpret=False):
    n = x.shape[0]

    def body(x_ref, degp_ref, bng_ref, bnb_ref, wp_ref, bp_ref, w1_ref,
             p1_ref, dinv_ref):
        deg = degp_ref[0:1, :] + degp_ref[1:2, :] + 1.0
        dinv = (1.0 / jnp.sqrt(deg)).reshape(_NACC, 1)[:n]
        h = _bn(x_ref[...], bng_ref[...], bnb_ref[...])
        h = jnp.maximum(_matT(h, wp_ref[...]) + bp_ref[...][None, :], 0.0)
        p1 = dinv * _matT(h, w1_ref[...])
        p1_ref[0, 0:n, :] = p1[:, 0:64]
        p1_ref[1, 0:n, :] = p1[:, 64:128]
        dinv_ref[...] = dinv

    return pl.pallas_call(
        body,
        out_shape=[jax.ShapeDtypeStruct((2, _NACC, 64), jnp.float32),
                   jax.ShapeDtypeStruct((n, 1), jnp.float32)],
        compiler_params=pltpu.CompilerParams(vmem_limit_bytes=100 * 1024 * 1024),
        interpret=interpret,
    )(x, degp, bng, bnb, wp, bp, w1)


def _tc_mid(sp, p1, dinv, b1, bng, bnb, w2, *, interpret=False):
    n = dinv.shape[0]

    def body(sp_ref, p1_ref, dinv_ref, b1_ref, bng_ref, bnb_ref, w2_ref, p2_ref):
        s = jnp.concatenate([sp_ref[0, :n, :], sp_ref[1, :n, :]], axis=1)
        p1 = jnp.concatenate([p1_ref[0, 0:n, :], p1_ref[1, 0:n, :]], axis=1)
        dinv = dinv_ref[...]
        out1 = dinv * (s + p1) + b1_ref[...][None, :]
        h1 = jnp.maximum(_bn(out1, bng_ref[...], bnb_ref[...]), 0.0)
        p2 = dinv * _matT(h1, w2_ref[...])
        p2_ref[0, 0:n, :] = p2[:, 0:64]
        p2_ref[1, 0:n, :] = p2[:, 64:128]

    return pl.pallas_call(
        body,
        out_shape=jax.ShapeDtypeStruct((2, _NACC, 64), jnp.float32),
        compiler_params=pltpu.CompilerParams(vmem_limit_bytes=100 * 1024 * 1024),
        interpret=interpret,
    )(sp, p1, dinv, b1, bng, bnb, w2)


def _tc_post(sp, p2, dinv, b2, bng, bnb, *, interpret=False):
    n = dinv.shape[0]

    def body(sp_ref, p2_ref, dinv_ref, b2_ref, bng_ref, bnb_ref, out_ref):
        s = jnp.concatenate([sp_ref[0, :n, :], sp_ref[1, :n, :]], axis=1)
        p2 = jnp.concatenate([p2_ref[0, 0:n, :], p2_ref[1, 0:n, :]], axis=1)
        out2 = dinv_ref[...] * (s + p2) + b2_ref[...][None, :]
        out_ref[...] = _bn(out2, bng_ref[...], bnb_ref[...])

    return pl.pallas_call(
        body,
        out_shape=jax.ShapeDtypeStruct((n, 128), jnp.float32),
        compiler_params=pltpu.CompilerParams(vmem_limit_bytes=100 * 1024 * 1024),
        interpret=interpret,
    )(sp, p2, dinv, b2, bng, bnb)


def kernel(x, edge_index, bn_in_g, bn_in_b, W_proj, b_proj, W1, b1,
           bn1_g, bn1_b, W2, b2, bn2_g, bn2_b):
    n = x.shape[0]
    e = edge_index.shape[1]
    pad = _EPAD - e
    src_p = jnp.concatenate([edge_index[0], jnp.zeros((pad,), edge_index.dtype)])
    dst_p = jnp.concatenate([edge_index[1], jnp.full((pad,), _DUMMY, edge_index.dtype)])
    src3 = src_p.reshape(_NW, _CPT, _CH)
    dst3 = dst_p.reshape(_NW, _CPT, _CH)
    src3s = src_p.reshape(_NS, _CPS, _CH)
    dst3s = dst_p.reshape(_NS, _CPS, _CH)
    ones_row = jnp.ones((_CH,), jnp.float32)
    zeros_col = jnp.zeros((_NACC // _NS,), jnp.float32)
    zeros_blk = jnp.zeros((_CH, 64), jnp.float32)

    degp = _sc_degree(dst3, ones_row, zeros_col)
    p1, dinv = _tc_pre(x, degp, bn_in_g, bn_in_b, W_proj, b_proj, W1)
    s1p = _sc_scatter(p1, src3s, dst3s, zeros_blk)
    p2 = _tc_mid(s1p, p1, dinv, b1, bn1_g, bn1_b, W2)
    s2p = _sc_scatter(p2, src3s, dst3s, zeros_blk)
    return _tc_post(s2p, p2, dinv, b2, bn2_g, bn2_b)


# speedup vs baseline: 17.2084x; 1.0566x over previous
import functools

import jax
import jax.numpy as jnp
from jax import lax
from jax.experimental import pallas as pl
from jax.experimental.pallas import tpu as pltpu
from jax.experimental.pallas import tpu_sc as plsc

_NC, _NS, _L = 2, 16, 16
_NW = _NC * _NS
_CH = 128
_CPT = 80
_CPS = 160
_BNA, _BNB = 3, 2
_EPAD = _NW * _CPT * _CH
_NACC = 10240
_DUMMY = 10016


def _mesh():
    return plsc.VectorSubcoreMesh(core_axis_name="c", subcore_axis_name="s",
                                  num_cores=_NC, num_subcores=_NS)


def _sc_degree(dst3, ones_row, zeros_col):
    n_tile = _NACC // _NS

    @functools.partial(
        pl.kernel,
        out_type=jax.ShapeDtypeStruct((_NC, _NACC), jnp.float32),
        mesh=_mesh(),
        scratch_types=[
            pltpu.VMEM((_CPT, _CH), jnp.int32),
            pltpu.VMEM((_CH,), jnp.float32),
            pltpu.VMEM((n_tile,), jnp.float32),
            pltpu.VMEM_SHARED((_NACC,), jnp.float32),
        ],
    )
    def k(dst_hbm, ones_hbm, zeros_hbm, out_hbm, idx_v, ones_v, stage_v, acc_sh):
        cid = lax.axis_index("c")
        sid = lax.axis_index("s")
        wid = cid * _NS + sid
        pltpu.sync_copy(ones_hbm, ones_v)
        pltpu.sync_copy(zeros_hbm, stage_v)
        pltpu.sync_copy(stage_v, acc_sh.at[pl.ds(sid * n_tile, n_tile)])
        pltpu.sync_copy(dst_hbm.at[wid], idx_v)
        plsc.subcore_barrier()

        def body(c, carry):
            pltpu.sync_copy(ones_v, acc_sh.at[idx_v.at[c]], add=True)
            return carry

        lax.fori_loop(0, _CPT, body, 0)
        plsc.subcore_barrier()
        pltpu.sync_copy(acc_sh.at[pl.ds(sid * n_tile, n_tile)], stage_v)
        pltpu.sync_copy(stage_v, out_hbm.at[cid, pl.ds(sid * n_tile, n_tile)])

    return k(dst3, ones_row, zeros_col)


def _sc_scatter(pstk, src3s, dst3s, zeros_blk):
    n_tile = _NACC // _NS
    n_stride = _BNA + _BNB
    n_grp = _CPS // n_stride

    @functools.partial(
        pl.kernel,
        out_type=jax.ShapeDtypeStruct((_NC, _NACC, 64), jnp.float32),
        mesh=_mesh(),
        compiler_params=pltpu.CompilerParams(use_tc_tiling_on_sc=False),
        scratch_types=[
            [pltpu.VMEM((_BNA, _CH), jnp.int32), pltpu.VMEM((_BNA, _CH), jnp.int32),
             pltpu.VMEM((_BNB, _CH), jnp.int32), pltpu.VMEM((_BNB, _CH), jnp.int32)],
            [pltpu.VMEM((_CH, 64), jnp.float32) for _ in range(_BNA + _BNB)],
            pltpu.VMEM_SHARED((_NACC, 64), jnp.float32),
            pltpu.VMEM_SHARED((_NACC, 64), jnp.float32),
            [pltpu.SemaphoreType.DMA for _ in range(4)],
        ],
    )
    def k(p_hbm, src_hbm, dst_hbm, zeros_hbm, out_hbm,
          idx_v, bufs, acc_sh, p_sh, sems):
        cid = lax.axis_index("c")
        sid = lax.axis_index("s")
        src_a, dst_a, src_b, dst_b = idx_v
        gsem_a, ssem_a, gsem_b, ssem_b = sems
        bufs_a, bufs_b = bufs[:_BNA], bufs[_BNA:]
        pltpu.sync_copy(zeros_hbm, acc_sh.at[pl.ds(sid * n_tile, n_tile)])
        pltpu.sync_copy(p_hbm.at[cid, pl.ds(sid * n_tile, n_tile)],
                        p_sh.at[pl.ds(sid * n_tile, n_tile)])
        plsc.subcore_barrier()

        def stage(g, off, nb, src_v, dst_v):
            pltpu.sync_copy(src_hbm.at[sid, pl.ds(g * n_stride + off, nb)],
                            src_v)
            pltpu.sync_copy(dst_hbm.at[sid, pl.ds(g * n_stride + off, nb)],
                            dst_v)

        def fire_g(src_v, bank, sem):
            return [pltpu.async_copy(p_sh.at[src_v.at[kk]], bank[kk], sem)
                    for kk in range(len(bank))]

        def fire_s(dst_v, bank, sem):
            return [pltpu.async_copy(bank[kk], acc_sh.at[dst_v.at[kk]], sem,
                                     add=True)
                    for kk in range(len(bank))]

        def wait_g(src_v, bank, sem):
            for kk in range(len(bank)):
                pltpu.make_async_copy(p_sh.at[src_v.at[kk]], bank[kk],
                                      sem).wait()

        def drain(ds):
            for d in ds:
                d.wait()

        stage(0, 0, _BNA, src_a, dst_a)
        stage(0, _BNA, _BNB, src_b, dst_b)
        fire_g(src_a, bufs_a, gsem_a)

        def group(g, carry):
            gb = fire_g(src_b, bufs_b, gsem_b)
            wait_g(src_a, bufs_a, gsem_a)
            drain(fire_s(dst_a, bufs_a, ssem_a))
            stage(g + 1, 0, _BNA, src_a, dst_a)
            fire_g(src_a, bufs_a, gsem_a)
            drain(gb)
            drain(fire_s(dst_b, bufs_b, ssem_b))
            stage(g + 1, _BNA, _BNB, src_b, dst_b)
            return carry

        lax.fori_loop(0, n_grp - 1, group, 0)
        gb = fire_g(src_b, bufs_b, gsem_b)
        wait_g(src_a, bufs_a, gsem_a)
        drain(fire_s(dst_a, bufs_a, ssem_a))
        drain(gb)
        drain(fire_s(dst_b, bufs_b, ssem_b))
        plsc.subcore_barrier()
        pltpu.sync_copy(acc_sh.at[pl.ds(sid * n_tile, n_tile)],
                        out_hbm.at[cid, pl.ds(sid * n_tile, n_tile)])

    return k(pstk, src3s, dst3s, zeros_blk)


def _bn(h, g, b):
    m = jnp.mean(h, axis=0, keepdims=True)
    c = h - m
    v = jnp.mean(c * c, axis=0, keepdims=True)
    return c / jnp.sqrt(v + 1e-5) * g[None, :] + b[None, :]


def _matT(h, w):
    return lax.dot_general(h, w, (((1,), (1,)), ((), ())),
                           precision=lax.Precision.HIGHEST,
                           preferred_element_type=jnp.float32)


def _tc_pre(x, degp, bng, bnb, wp, bp, w1, *, interpret=False):
    n = x.shape[0]

    def body(x_ref, degp_ref, bng_ref, bnb_ref, wp_ref, bp_ref, w1_ref,
             p1_ref, dinv_ref):
        deg = degp_ref[0:1, :] + degp_ref[1:2, :] + 1.0
        dinv = (1.0 / jnp.sqrt(deg)).reshape(_NACC, 1)[:n]
        h = _bn(x_ref[...], bng_ref[...], bnb_ref[...])
        h = jnp.maximum(_matT(h, wp_ref[...]) + bp_ref[...][None, :], 0.0)
        p1 = dinv * _matT(h, w1_ref[...])
        p1_ref[0, 0:n, :] = p1[:, 0:64]
        p1_ref[1, 0:n, :] = p1[:, 64:128]
        dinv_ref[...] = dinv

    return pl.pallas_call(
        body,
        out_shape=[jax.ShapeDtypeStruct((2, _NACC, 64), jnp.float32),
                   jax.ShapeDtypeStruct((n, 1), jnp.float32)],
        compiler_params=pltpu.CompilerParams(vmem_limit_bytes=100 * 1024 * 1024),
        interpret=interpret,
    )(x, degp, bng, bnb, wp, bp, w1)


def _tc_mid(sp, p1, dinv, b1, bng, bnb, w2, *, interpret=False):
    n = dinv.shape[0]

    def body(sp_ref, p1_ref, dinv_ref, b1_ref, bng_ref, bnb_ref, w2_ref, p2_ref):
        s = jnp.concatenate([sp_ref[0, :n, :], sp_ref[1, :n, :]], axis=1)
        p1 = jnp.concatenate([p1_ref[0, 0:n, :], p1_ref[1, 0:n, :]], axis=1)
        dinv = dinv_ref[...]
        out1 = dinv * (s + p1) + b1_ref[...][None, :]
        h1 = jnp.maximum(_bn(out1, bng_ref[...], bnb_ref[...]), 0.0)
        p2 = dinv * _matT(h1, w2_ref[...])
        p2_ref[0, 0:n, :] = p2[:, 0:64]
        p2_ref[1, 0:n, :] = p2[:, 64:128]

    return pl.pallas_call(
        body,
        out_shape=jax.ShapeDtypeStruct((2, _NACC, 64), jnp.float32),
        compiler_params=pltpu.CompilerParams(vmem_limit_bytes=100 * 1024 * 1024),
        interpret=interpret,
    )(sp, p1, dinv, b1, bng, bnb, w2)


def _tc_post(sp, p2, dinv, b2, bng, bnb, *, interpret=False):
    n = dinv.shape[0]

    def body(sp_ref, p2_ref, dinv_ref, b2_ref, bng_ref, bnb_ref, out_ref):
        s = jnp.concatenate([sp_ref[0, :n, :], sp_ref[1, :n, :]], axis=1)
        p2 = jnp.concatenate([p2_ref[0, 0:n, :], p2_ref[1, 0:n, :]], axis=1)
        out2 = dinv_ref[...] * (s + p2) + b2_ref[...][None, :]
        out_ref[...] = _bn(out2, bng_ref[...], bnb_ref[...])

    return pl.pallas_call(
        body,
        out_shape=jax.ShapeDtypeStruct((n, 128), jnp.float32),
        compiler_params=pltpu.CompilerParams(vmem_limit_bytes=100 * 1024 * 1024),
        interpret=interpret,
    )(sp, p2, dinv, b2, bng, bnb)


def kernel(x, edge_index, bn_in_g, bn_in_b, W_proj, b_proj, W1, b1,
           bn1_g, bn1_b, W2, b2, bn2_g, bn2_b):
    n = x.shape[0]
    e = edge_index.shape[1]
    pad = _EPAD - e
    src_p = jnp.concatenate([edge_index[0], jnp.zeros((pad,), edge_index.dtype)])
    dst_p = jnp.concatenate([edge_index[1], jnp.full((pad,), _DUMMY, edge_index.dtype)])
    src3 = src_p.reshape(_NW, _CPT, _CH)
    dst3 = dst_p.reshape(_NW, _CPT, _CH)
    src3s = src_p.reshape(_NS, _CPS, _CH)
    dst3s = dst_p.reshape(_NS, _CPS, _CH)
    ones_row = jnp.ones((_CH,), jnp.float32)
    zeros_col = jnp.zeros((_NACC // _NS,), jnp.float32)
    zeros_blk = jnp.zeros((_NACC // _NS, 64), jnp.float32)

    degp = _sc_degree(dst3, ones_row, zeros_col)
    p1, dinv = _tc_pre(x, degp, bn_in_g, bn_in_b, W_proj, b_proj, W1)
    s1p = _sc_scatter(p1, src3s, dst3s, zeros_blk)
    p2 = _tc_mid(s1p, p1, dinv, b1, bn1_g, bn1_b, W2)
    s2p = _sc_scatter(p2, src3s, dst3s, zeros_blk)
    return _tc_post(s2p, p2, dinv, b2, bn2_g, bn2_b)


# speedup vs baseline: 20.6641x; 1.2008x over previous
import functools

import jax
import jax.numpy as jnp
from jax import lax
from jax.experimental import pallas as pl
from jax.experimental.pallas import tpu as pltpu
from jax.experimental.pallas import tpu_sc as plsc

_NC, _NS, _L = 2, 16, 16
_NW = _NC * _NS
_CH = 128
_CPT = 80
_CPS = 160
_BNA, _BNB = 3, 2
_EPAD = _NW * _CPT * _CH
_NACC = 10240
_DUMMY = 10016


def _mesh():
    return plsc.VectorSubcoreMesh(core_axis_name="c", subcore_axis_name="s",
                                  num_cores=_NC, num_subcores=_NS)


def _sc_degree(dst3, ones_row, zeros_col):
    n_tile = _NACC // _NS

    @functools.partial(
        pl.kernel,
        out_type=jax.ShapeDtypeStruct((_NC, _NACC), jnp.float32),
        mesh=_mesh(),
        scratch_types=[
            pltpu.VMEM((_CPT, _CH), jnp.int32),
            pltpu.VMEM((_CH,), jnp.float32),
            pltpu.VMEM((n_tile,), jnp.float32),
            pltpu.VMEM_SHARED((_NACC,), jnp.float32),
        ],
    )
    def k(dst_hbm, ones_hbm, zeros_hbm, out_hbm, idx_v, ones_v, stage_v, acc_sh):
        cid = lax.axis_index("c")
        sid = lax.axis_index("s")
        wid = cid * _NS + sid
        pltpu.sync_copy(ones_hbm, ones_v)
        pltpu.sync_copy(zeros_hbm, stage_v)
        pltpu.sync_copy(stage_v, acc_sh.at[pl.ds(sid * n_tile, n_tile)])
        pltpu.sync_copy(dst_hbm.at[wid], idx_v)
        plsc.subcore_barrier()

        def body(c, carry):
            pltpu.sync_copy(ones_v, acc_sh.at[idx_v.at[c]], add=True)
            return carry

        lax.fori_loop(0, _CPT, body, 0)
        plsc.subcore_barrier()
        pltpu.sync_copy(acc_sh.at[pl.ds(sid * n_tile, n_tile)], stage_v)
        pltpu.sync_copy(stage_v, out_hbm.at[cid, pl.ds(sid * n_tile, n_tile)])

    return k(dst3, ones_row, zeros_col)


def _sc_scatter(pstk, src3s, dst3s, zeros_blk):
    n_tile = _NACC // _NS
    n_stride = _BNA + _BNB
    n_grp = _CPS // n_stride

    @functools.partial(
        pl.kernel,
        out_type=jax.ShapeDtypeStruct((_NC, _NACC, 64), jnp.float32),
        mesh=_mesh(),
        compiler_params=pltpu.CompilerParams(use_tc_tiling_on_sc=False),
        scratch_types=[
            [pltpu.VMEM((_BNA, _CH), jnp.int32), pltpu.VMEM((_BNA, _CH), jnp.int32),
             pltpu.VMEM((_BNB, _CH), jnp.int32), pltpu.VMEM((_BNB, _CH), jnp.int32)],
            [pltpu.VMEM((_CH, 64), jnp.float32) for _ in range(_BNA + _BNB)],
            pltpu.VMEM_SHARED((_NACC, 64), jnp.float32),
            pltpu.VMEM_SHARED((_NACC, 64), jnp.float32),
            [pltpu.SemaphoreType.DMA for _ in range(6)],
        ],
    )
    def k(p_hbm, src_hbm, dst_hbm, zeros_hbm, out_hbm,
          idx_v, bufs, acc_sh, p_sh, sems):
        cid = lax.axis_index("c")
        sid = lax.axis_index("s")
        src_a, dst_a, src_b, dst_b = idx_v
        gsem_a, ssem_a, gsem_b, ssem_b, isem_a, isem_b = sems
        bufs_a, bufs_b = bufs[:_BNA], bufs[_BNA:]
        pltpu.sync_copy(zeros_hbm, acc_sh.at[pl.ds(sid * n_tile, n_tile)])
        pltpu.sync_copy(p_hbm.at[cid, pl.ds(sid * n_tile, n_tile)],
                        p_sh.at[pl.ds(sid * n_tile, n_tile)])
        plsc.subcore_barrier()

        def stage(g, off, nb, src_v, dst_v):
            pltpu.sync_copy(src_hbm.at[sid, pl.ds(g * n_stride + off, nb)],
                            src_v)
            pltpu.sync_copy(dst_hbm.at[sid, pl.ds(g * n_stride + off, nb)],
                            dst_v)

        def stage_async(g, off, nb, hbm, dst_v, sem):
            pltpu.async_copy(hbm.at[sid, pl.ds(g * n_stride + off, nb)],
                             dst_v, sem)

        def stage_wait(g, off, nb, hbm, dst_v, sem):
            pltpu.make_async_copy(hbm.at[sid, pl.ds(g * n_stride + off, nb)],
                                  dst_v, sem).wait()

        def fire_g(src_v, bank, sem):
            return [pltpu.async_copy(p_sh.at[src_v.at[kk]], bank[kk], sem)
                    for kk in range(len(bank))]

        def fire_s(dst_v, bank, sem):
            return [pltpu.async_copy(bank[kk], acc_sh.at[dst_v.at[kk]], sem,
                                     add=True)
                    for kk in range(len(bank))]

        def wait_g(src_v, bank, sem):
            for kk in range(len(bank)):
                pltpu.make_async_copy(p_sh.at[src_v.at[kk]], bank[kk],
                                      sem).wait()

        def drain(ds):
            for d in ds:
                d.wait()

        stage(0, 0, _BNA, src_a, dst_a)
        stage_async(0, _BNA, _BNB, src_hbm, src_b, isem_b)
        stage_async(0, _BNA, _BNB, dst_hbm, dst_b, isem_b)
        fire_g(src_a, bufs_a, gsem_a)

        def group(g, carry):
            stage_wait(g, _BNA, _BNB, src_hbm, src_b, isem_b)
            stage_wait(g, _BNA, _BNB, dst_hbm, dst_b, isem_b)
            gb = fire_g(src_b, bufs_b, gsem_b)
            wait_g(src_a, bufs_a, gsem_a)
            stage_async(g + 1, 0, _BNA, src_hbm, src_a, isem_a)
            drain(fire_s(dst_a, bufs_a, ssem_a))
            stage_async(g + 1, 0, _BNA, dst_hbm, dst_a, isem_a)
            stage_wait(g + 1, 0, _BNA, src_hbm, src_a, isem_a)
            stage_wait(g + 1, 0, _BNA, dst_hbm, dst_a, isem_a)
            fire_g(src_a, bufs_a, gsem_a)
            drain(gb)
            drain(fire_s(dst_b, bufs_b, ssem_b))
            stage_async(g + 1, _BNA, _BNB, src_hbm, src_b, isem_b)
            stage_async(g + 1, _BNA, _BNB, dst_hbm, dst_b, isem_b)
            return carry

        lax.fori_loop(0, n_grp - 1, group, 0)
        stage_wait(n_grp - 1, _BNA, _BNB, src_hbm, src_b, isem_b)
        stage_wait(n_grp - 1, _BNA, _BNB, dst_hbm, dst_b, isem_b)
        gb = fire_g(src_b, bufs_b, gsem_b)
        wait_g(src_a, bufs_a, gsem_a)
        drain(fire_s(dst_a, bufs_a, ssem_a))
        drain(gb)
        drain(fire_s(dst_b, bufs_b, ssem_b))
        plsc.subcore_barrier()
        pltpu.sync_copy(acc_sh.at[pl.ds(sid * n_tile, n_tile)],
                        out_hbm.at[cid, pl.ds(sid * n_tile, n_tile)])

    return k(pstk, src3s, dst3s, zeros_blk)


def _bn(h, g, b):
    m = jnp.mean(h, axis=0, keepdims=True)
    c = h - m
    v = jnp.mean(c * c, axis=0, keepdims=True)
    return c / jnp.sqrt(v + 1e-5) * g[None, :] + b[None, :]


def _matT(h, w):
    return lax.dot_general(h, w, (((1,), (1,)), ((), ())),
                           precision=lax.Precision.HIGHEST,
                           preferred_element_type=jnp.float32)


def _tc_pre(x, degp, bng, bnb, wp, bp, w1, *, interpret=False):
    n = x.shape[0]

    def body(x_ref, degp_ref, bng_ref, bnb_ref, wp_ref, bp_ref, w1_ref,
             p1_ref, dinv_ref):
        deg = degp_ref[0:1, :] + degp_ref[1:2, :] + 1.0
        dinv = (1.0 / jnp.sqrt(deg)).reshape(_NACC, 1)[:n]
        h = _bn(x_ref[...], bng_ref[...], bnb_ref[...])
        h = jnp.maximum(_matT(h, wp_ref[...]) + bp_ref[...][None, :], 0.0)
        p1 = dinv * _matT(h, w1_ref[...])
        p1_ref[0, 0:n, :] = p1[:, 0:64]
        p1_ref[1, 0:n, :] = p1[:, 64:128]
        dinv_ref[...] = dinv

    return pl.pallas_call(
        body,
        out_shape=[jax.ShapeDtypeStruct((2, _NACC, 64), jnp.float32),
                   jax.ShapeDtypeStruct((n, 1), jnp.float32)],
        compiler_params=pltpu.CompilerParams(vmem_limit_bytes=100 * 1024 * 1024),
        interpret=interpret,
    )(x, degp, bng, bnb, wp, bp, w1)


def _tc_mid(sp, p1, dinv, b1, bng, bnb, w2, *, interpret=False):
    n = dinv.shape[0]

    def body(sp_ref, p1_ref, dinv_ref, b1_ref, bng_ref, bnb_ref, w2_ref, p2_ref):
        s = jnp.concatenate([sp_ref[0, :n, :], sp_ref[1, :n, :]], axis=1)
        p1 = jnp.concatenate([p1_ref[0, 0:n, :], p1_ref[1, 0:n, :]], axis=1)
        dinv = dinv_ref[...]
        out1 = dinv * (s + p1) + b1_ref[...][None, :]
        h1 = jnp.maximum(_bn(out1, bng_ref[...], bnb_ref[...]), 0.0)
        p2 = dinv * _matT(h1, w2_ref[...])
        p2_ref[0, 0:n, :] = p2[:, 0:64]
        p2_ref[1, 0:n, :] = p2[:, 64:128]

    return pl.pallas_call(
        body,
        out_shape=jax.ShapeDtypeStruct((2, _NACC, 64), jnp.float32),
        compiler_params=pltpu.CompilerParams(vmem_limit_bytes=100 * 1024 * 1024),
        interpret=interpret,
    )(sp, p1, dinv, b1, bng, bnb, w2)


def _tc_post(sp, p2, dinv, b2, bng, bnb, *, interpret=False):
    n = dinv.shape[0]

    def body(sp_ref, p2_ref, dinv_ref, b2_ref, bng_ref, bnb_ref, out_ref):
        s = jnp.concatenate([sp_ref[0, :n, :], sp_ref[1, :n, :]], axis=1)
        p2 = jnp.concatenate([p2_ref[0, 0:n, :], p2_ref[1, 0:n, :]], axis=1)
        out2 = dinv_ref[...] * (s + p2) + b2_ref[...][None, :]
        out_ref[...] = _bn(out2, bng_ref[...], bnb_ref[...])

    return pl.pallas_call(
        body,
        out_shape=jax.ShapeDtypeStruct((n, 128), jnp.float32),
        compiler_params=pltpu.CompilerParams(vmem_limit_bytes=100 * 1024 * 1024),
        interpret=interpret,
    )(sp, p2, dinv, b2, bng, bnb)


def kernel(x, edge_index, bn_in_g, bn_in_b, W_proj, b_proj, W1, b1,
           bn1_g, bn1_b, W2, b2, bn2_g, bn2_b):
    n = x.shape[0]
    e = edge_index.shape[1]
    pad = _EPAD - e
    src_p = jnp.concatenate([edge_index[0], jnp.zeros((pad,), edge_index.dtype)])
    dst_p = jnp.concatenate([edge_index[1], jnp.full((pad,), _DUMMY, edge_index.dtype)])
    src3 = src_p.reshape(_NW, _CPT, _CH)
    dst3 = dst_p.reshape(_NW, _CPT, _CH)
    src3s = src_p.reshape(_NS, _CPS, _CH)
    dst3s = dst_p.reshape(_NS, _CPS, _CH)
    ones_row = jnp.ones((_CH,), jnp.float32)
    zeros_col = jnp.zeros((_NACC // _NS,), jnp.float32)
    zeros_blk = jnp.zeros((_NACC // _NS, 64), jnp.float32)

    degp = _sc_degree(dst3, ones_row, zeros_col)
    p1, dinv = _tc_pre(x, degp, bn_in_g, bn_in_b, W_proj, b_proj, W1)
    s1p = _sc_scatter(p1, src3s, dst3s, zeros_blk)
    p2 = _tc_mid(s1p, p1, dinv, b1, bn1_g, bn1_b, W2)
    s2p = _sc_scatter(p2, src3s, dst3s, zeros_blk)
    return _tc_post(s2p, p2, dinv, b2, bn2_g, bn2_b)


# speedup vs baseline: 22.1048x; 1.0697x over previous
import functools

import jax
import jax.numpy as jnp
from jax import lax
from jax.experimental import pallas as pl
from jax.experimental.pallas import tpu as pltpu
from jax.experimental.pallas import tpu_sc as plsc

_NC, _NS, _L = 2, 16, 16
_NW = _NC * _NS
_CH = 128
_CPT = 80
_CPS = 160
_BNA, _BNB = 3, 2
_EPAD = _NW * _CPT * _CH
_NACC = 10240
_DUMMY = 10016


def _mesh():
    return plsc.VectorSubcoreMesh(core_axis_name="c", subcore_axis_name="s",
                                  num_cores=_NC, num_subcores=_NS)


def _sc_degree(dst3, ones_row, zeros_col):
    n_tile = _NACC // _NS

    @functools.partial(
        pl.kernel,
        out_type=jax.ShapeDtypeStruct((_NC, _NACC), jnp.float32),
        mesh=_mesh(),
        scratch_types=[
            pltpu.VMEM((_CPT, _CH), jnp.int32),
            pltpu.VMEM((_CH,), jnp.float32),
            pltpu.VMEM((n_tile,), jnp.float32),
            pltpu.VMEM_SHARED((_NACC,), jnp.float32),
        ],
    )
    def k(dst_hbm, ones_hbm, zeros_hbm, out_hbm, idx_v, ones_v, stage_v, acc_sh):
        cid = lax.axis_index("c")
        sid = lax.axis_index("s")
        wid = cid * _NS + sid
        pltpu.sync_copy(ones_hbm, ones_v)
        pltpu.sync_copy(zeros_hbm, stage_v)
        pltpu.sync_copy(stage_v, acc_sh.at[pl.ds(sid * n_tile, n_tile)])
        pltpu.sync_copy(dst_hbm.at[wid], idx_v)
        plsc.subcore_barrier()

        def body(c, carry):
            pltpu.sync_copy(ones_v, acc_sh.at[idx_v.at[c]], add=True)
            return carry

        lax.fori_loop(0, _CPT, body, 0)
        plsc.subcore_barrier()
        pltpu.sync_copy(acc_sh.at[pl.ds(sid * n_tile, n_tile)], stage_v)
        pltpu.sync_copy(stage_v, out_hbm.at[cid, pl.ds(sid * n_tile, n_tile)])

    return k(dst3, ones_row, zeros_col)


def _sc_scatter(pstk, src3s, dst3s, zeros_blk):
    n_tile = _NACC // _NS
    n_stride = _BNA + _BNB
    n_grp = _CPS // n_stride

    @functools.partial(
        pl.kernel,
        out_type=jax.ShapeDtypeStruct((_NC, _NACC, 64), jnp.float32),
        mesh=_mesh(),
        compiler_params=pltpu.CompilerParams(use_tc_tiling_on_sc=False),
        scratch_types=[
            [pltpu.VMEM((_BNA, _CH), jnp.int32), pltpu.VMEM((_BNA, _CH), jnp.int32),
             pltpu.VMEM((_BNB, _CH), jnp.int32), pltpu.VMEM((_BNB, _CH), jnp.int32)],
            [pltpu.VMEM((_CH, 64), jnp.float32) for _ in range(_BNA + _BNB)],
            pltpu.VMEM_SHARED((_NACC, 64), jnp.float32),
            pltpu.VMEM_SHARED((_NACC, 64), jnp.float32),
            [pltpu.SemaphoreType.DMA for _ in range(6)],
        ],
    )
    def k(p_hbm, src_hbm, dst_hbm, zeros_hbm, out_hbm,
          idx_v, bufs, acc_sh, p_sh, sems):
        cid = lax.axis_index("c")
        sid = lax.axis_index("s")
        src_a, dst_a, src_b, dst_b = idx_v
        gsem_a, ssem_a, gsem_b, ssem_b, isem_a, isem_b = sems
        bufs_a, bufs_b = bufs[:_BNA], bufs[_BNA:]
        pltpu.sync_copy(zeros_hbm, acc_sh.at[pl.ds(sid * n_tile, n_tile)])
        pltpu.sync_copy(p_hbm.at[pl.ds(sid * n_tile, n_tile),
                                 pl.ds(cid * 64, 64)],
                        p_sh.at[pl.ds(sid * n_tile, n_tile)])
        plsc.subcore_barrier()

        def stage(g, off, nb, src_v, dst_v):
            pltpu.sync_copy(src_hbm.at[sid, pl.ds(g * n_stride + off, nb)],
                            src_v)
            pltpu.sync_copy(dst_hbm.at[sid, pl.ds(g * n_stride + off, nb)],
                            dst_v)

        def stage_async(g, off, nb, hbm, dst_v, sem):
            pltpu.async_copy(hbm.at[sid, pl.ds(g * n_stride + off, nb)],
                             dst_v, sem)

        def stage_wait(g, off, nb, hbm, dst_v, sem):
            pltpu.make_async_copy(hbm.at[sid, pl.ds(g * n_stride + off, nb)],
                                  dst_v, sem).wait()

        def fire_g(src_v, bank, sem):
            return [pltpu.async_copy(p_sh.at[src_v.at[kk]], bank[kk], sem)
                    for kk in range(len(bank))]

        def fire_s(dst_v, bank, sem):
            return [pltpu.async_copy(bank[kk], acc_sh.at[dst_v.at[kk]], sem,
                                     add=True)
                    for kk in range(len(bank))]

        def wait_g(src_v, bank, sem):
            for kk in range(len(bank)):
                pltpu.make_async_copy(p_sh.at[src_v.at[kk]], bank[kk],
                                      sem).wait()

        def drain(ds):
            for d in ds:
                d.wait()

        stage(0, 0, _BNA, src_a, dst_a)
        stage_async(0, _BNA, _BNB, src_hbm, src_b, isem_b)
        stage_async(0, _BNA, _BNB, dst_hbm, dst_b, isem_b)
        fire_g(src_a, bufs_a, gsem_a)

        def group(g, carry):
            stage_wait(g, _BNA, _BNB, src_hbm, src_b, isem_b)
            stage_wait(g, _BNA, _BNB, dst_hbm, dst_b, isem_b)
            gb = fire_g(src_b, bufs_b, gsem_b)
            wait_g(src_a, bufs_a, gsem_a)
            stage_async(g + 1, 0, _BNA, src_hbm, src_a, isem_a)
            drain(fire_s(dst_a, bufs_a, ssem_a))
            stage_async(g + 1, 0, _BNA, dst_hbm, dst_a, isem_a)
            stage_wait(g + 1, 0, _BNA, src_hbm, src_a, isem_a)
            stage_wait(g + 1, 0, _BNA, dst_hbm, dst_a, isem_a)
            fire_g(src_a, bufs_a, gsem_a)
            drain(gb)
            drain(fire_s(dst_b, bufs_b, ssem_b))
            stage_async(g + 1, _BNA, _BNB, src_hbm, src_b, isem_b)
            stage_async(g + 1, _BNA, _BNB, dst_hbm, dst_b, isem_b)
            return carry

        lax.fori_loop(0, n_grp - 1, group, 0)
        stage_wait(n_grp - 1, _BNA, _BNB, src_hbm, src_b, isem_b)
        stage_wait(n_grp - 1, _BNA, _BNB, dst_hbm, dst_b, isem_b)
        gb = fire_g(src_b, bufs_b, gsem_b)
        wait_g(src_a, bufs_a, gsem_a)
        drain(fire_s(dst_a, bufs_a, ssem_a))
        drain(gb)
        drain(fire_s(dst_b, bufs_b, ssem_b))
        plsc.subcore_barrier()
        pltpu.sync_copy(acc_sh.at[pl.ds(sid * n_tile, n_tile)],
                        out_hbm.at[cid, pl.ds(sid * n_tile, n_tile)])

    return k(pstk, src3s, dst3s, zeros_blk)


def _bn(h, g, b):
    m = jnp.mean(h, axis=0, keepdims=True)
    c = h - m
    v = jnp.mean(c * c, axis=0, keepdims=True)
    return c / jnp.sqrt(v + 1e-5) * g[None, :] + b[None, :]


def _matT(h, w):
    return lax.dot_general(h, w, (((1,), (1,)), ((), ())),
                           precision=lax.Precision.HIGHEST,
                           preferred_element_type=jnp.float32)


def _tc_pre(x, degp, bng, bnb, wp, bp, w1, *, interpret=False):
    n = x.shape[0]

    def body(x_ref, degp_ref, bng_ref, bnb_ref, wp_ref, bp_ref, w1_ref,
             p1_ref, dinv_ref):
        deg = degp_ref[0:1, :] + degp_ref[1:2, :] + 1.0
        dinv = (1.0 / jnp.sqrt(deg)).reshape(_NACC, 1)[:n]
        h = _bn(x_ref[...], bng_ref[...], bnb_ref[...])
        h = jnp.maximum(_matT(h, wp_ref[...]) + bp_ref[...][None, :], 0.0)
        p1_ref[0:n, :] = dinv * _matT(h, w1_ref[...])
        dinv_ref[...] = dinv

    return pl.pallas_call(
        body,
        out_shape=[jax.ShapeDtypeStruct((_NACC, 128), jnp.float32),
                   jax.ShapeDtypeStruct((n, 1), jnp.float32)],
        compiler_params=pltpu.CompilerParams(vmem_limit_bytes=100 * 1024 * 1024),
        interpret=interpret,
    )(x, degp, bng, bnb, wp, bp, w1)


def _tc_mid(sp, p1, dinv, b1, bng, bnb, w2, *, interpret=False):
    n = dinv.shape[0]

    def body(sp_ref, p1_ref, dinv_ref, b1_ref, bng_ref, bnb_ref, w2_ref, p2_ref):
        s = jnp.concatenate([sp_ref[0, :n, :], sp_ref[1, :n, :]], axis=1)
        dinv = dinv_ref[...]
        out1 = dinv * (s + p1_ref[0:n, :]) + b1_ref[...][None, :]
        h1 = jnp.maximum(_bn(out1, bng_ref[...], bnb_ref[...]), 0.0)
        p2_ref[0:n, :] = dinv * _matT(h1, w2_ref[...])

    return pl.pallas_call(
        body,
        out_shape=jax.ShapeDtypeStruct((_NACC, 128), jnp.float32),
        compiler_params=pltpu.CompilerParams(vmem_limit_bytes=100 * 1024 * 1024),
        interpret=interpret,
    )(sp, p1, dinv, b1, bng, bnb, w2)


def _tc_post(sp, p2, dinv, b2, bng, bnb, *, interpret=False):
    n = dinv.shape[0]

    def body(sp_ref, p2_ref, dinv_ref, b2_ref, bng_ref, bnb_ref, out_ref):
        s = jnp.concatenate([sp_ref[0, :n, :], sp_ref[1, :n, :]], axis=1)
        out2 = dinv_ref[...] * (s + p2_ref[0:n, :]) + b2_ref[...][None, :]
        out_ref[...] = _bn(out2, bng_ref[...], bnb_ref[...])

    return pl.pallas_call(
        body,
        out_shape=jax.ShapeDtypeStruct((n, 128), jnp.float32),
        compiler_params=pltpu.CompilerParams(vmem_limit_bytes=100 * 1024 * 1024),
        interpret=interpret,
    )(sp, p2, dinv, b2, bng, bnb)


def kernel(x, edge_index, bn_in_g, bn_in_b, W_proj, b_proj, W1, b1,
           bn1_g, bn1_b, W2, b2, bn2_g, bn2_b):
    n = x.shape[0]
    e = edge_index.shape[1]
    pad = _EPAD - e
    src_p = jnp.concatenate([edge_index[0], jnp.zeros((pad,), edge_index.dtype)])
    dst_p = jnp.concatenate([edge_index[1], jnp.full((pad,), _DUMMY, edge_index.dtype)])
    src3 = src_p.reshape(_NW, _CPT, _CH)
    dst3 = dst_p.reshape(_NW, _CPT, _CH)
    src3s = src_p.reshape(_NS, _CPS, _CH)
    dst3s = dst_p.reshape(_NS, _CPS, _CH)
    ones_row = jnp.ones((_CH,), jnp.float32)
    zeros_col = jnp.zeros((_NACC // _NS,), jnp.float32)
    zeros_blk = jnp.zeros((_NACC // _NS, 64), jnp.float32)

    degp = _sc_degree(dst3, ones_row, zeros_col)
    p1, dinv = _tc_pre(x, degp, bn_in_g, bn_in_b, W_proj, b_proj, W1)
    s1p = _sc_scatter(p1, src3s, dst3s, zeros_blk)
    p2 = _tc_mid(s1p, p1, dinv, b1, bn1_g, bn1_b, W2)
    s2p = _sc_scatter(p2, src3s, dst3s, zeros_blk)
    return _tc_post(s2p, p2, dinv, b2, bn2_g, bn2_b)


# speedup vs baseline: 23.6079x; 1.0680x over previous
import functools

import jax
import jax.numpy as jnp
from jax import lax
from jax.experimental import pallas as pl
from jax.experimental.pallas import tpu as pltpu
from jax.experimental.pallas import tpu_sc as plsc

_NC, _NS, _L = 2, 16, 16
_NW = _NC * _NS
_CH = 128
_CPT = 80
_CPS = 160
_BNA, _BNB = 3, 2
_EPAD = _NW * _CPT * _CH
_NACC = 10240
_DUMMY = 10016


def _mesh():
    return plsc.VectorSubcoreMesh(core_axis_name="c", subcore_axis_name="s",
                                  num_cores=_NC, num_subcores=_NS)


def _sc_degree(dst3, ones_row, zeros_col):
    n_tile = _NACC // _NS

    @functools.partial(
        pl.kernel,
        out_type=jax.ShapeDtypeStruct((_NC, _NACC), jnp.float32),
        mesh=_mesh(),
        scratch_types=[
            pltpu.VMEM((_CPT, _CH), jnp.int32),
            pltpu.VMEM((_CH,), jnp.float32),
            pltpu.VMEM((n_tile,), jnp.float32),
            pltpu.VMEM_SHARED((_NACC,), jnp.float32),
        ],
    )
    def k(dst_hbm, ones_hbm, zeros_hbm, out_hbm, idx_v, ones_v, stage_v, acc_sh):
        cid = lax.axis_index("c")
        sid = lax.axis_index("s")
        wid = cid * _NS + sid
        pltpu.sync_copy(ones_hbm, ones_v)
        pltpu.sync_copy(zeros_hbm, stage_v)
        pltpu.sync_copy(stage_v, acc_sh.at[pl.ds(sid * n_tile, n_tile)])
        pltpu.sync_copy(dst_hbm.at[wid], idx_v)
        plsc.subcore_barrier()

        def body(c, carry):
            pltpu.sync_copy(ones_v, acc_sh.at[idx_v.at[c]], add=True)
            return carry

        lax.fori_loop(0, _CPT, body, 0)
        plsc.subcore_barrier()
        pltpu.sync_copy(acc_sh.at[pl.ds(sid * n_tile, n_tile)], stage_v)
        pltpu.sync_copy(stage_v, out_hbm.at[cid, pl.ds(sid * n_tile, n_tile)])

    return k(dst3, ones_row, zeros_col)


def _sc_scatter(pstk, src3s, dst3s, zeros_blk):
    n_tile = _NACC // _NS
    n_stride = _BNA + _BNB
    n_grp = _CPS // n_stride

    @functools.partial(
        pl.kernel,
        out_type=jax.ShapeDtypeStruct((_NACC, 128), jnp.float32),
        mesh=_mesh(),
        compiler_params=pltpu.CompilerParams(use_tc_tiling_on_sc=False),
        scratch_types=[
            [pltpu.VMEM((_BNA, _CH), jnp.int32), pltpu.VMEM((_BNA, _CH), jnp.int32),
             pltpu.VMEM((_BNB, _CH), jnp.int32), pltpu.VMEM((_BNB, _CH), jnp.int32)],
            [pltpu.VMEM((_CH, 64), jnp.float32) for _ in range(_BNA + _BNB)],
            pltpu.VMEM_SHARED((_NACC, 64), jnp.float32),
            pltpu.VMEM_SHARED((_NACC, 64), jnp.float32),
            [pltpu.SemaphoreType.DMA for _ in range(6)],
        ],
    )
    def k(p_hbm, src_hbm, dst_hbm, zeros_hbm, out_hbm,
          idx_v, bufs, acc_sh, p_sh, sems):
        cid = lax.axis_index("c")
        sid = lax.axis_index("s")
        src_a, dst_a, src_b, dst_b = idx_v
        gsem_a, ssem_a, gsem_b, ssem_b, isem_a, isem_b = sems
        bufs_a, bufs_b = bufs[:_BNA], bufs[_BNA:]
        pltpu.sync_copy(zeros_hbm, acc_sh.at[pl.ds(sid * n_tile, n_tile)])
        pltpu.sync_copy(p_hbm.at[pl.ds(sid * n_tile, n_tile),
                                 pl.ds(cid * 64, 64)],
                        p_sh.at[pl.ds(sid * n_tile, n_tile)])
        plsc.subcore_barrier()

        def stage(g, off, nb, src_v, dst_v):
            pltpu.sync_copy(src_hbm.at[sid, pl.ds(g * n_stride + off, nb)],
                            src_v)
            pltpu.sync_copy(dst_hbm.at[sid, pl.ds(g * n_stride + off, nb)],
                            dst_v)

        def stage_async(g, off, nb, hbm, dst_v, sem):
            pltpu.async_copy(hbm.at[sid, pl.ds(g * n_stride + off, nb)],
                             dst_v, sem)

        def stage_wait(g, off, nb, hbm, dst_v, sem):
            pltpu.make_async_copy(hbm.at[sid, pl.ds(g * n_stride + off, nb)],
                                  dst_v, sem).wait()

        def fire_g(src_v, bank, sem):
            return [pltpu.async_copy(p_sh.at[src_v.at[kk]], bank[kk], sem)
                    for kk in range(len(bank))]

        def fire_s(dst_v, bank, sem):
            return [pltpu.async_copy(bank[kk], acc_sh.at[dst_v.at[kk]], sem,
                                     add=True)
                    for kk in range(len(bank))]

        def wait_g(src_v, bank, sem):
            for kk in range(len(bank)):
                pltpu.make_async_copy(p_sh.at[src_v.at[kk]], bank[kk],
                                      sem).wait()

        def drain(ds):
            for d in ds:
                d.wait()

        stage(0, 0, _BNA, src_a, dst_a)
        stage_async(0, _BNA, _BNB, src_hbm, src_b, isem_b)
        stage_async(0, _BNA, _BNB, dst_hbm, dst_b, isem_b)
        fire_g(src_a, bufs_a, gsem_a)

        def group(g, carry):
            stage_wait(g, _BNA, _BNB, src_hbm, src_b, isem_b)
            stage_wait(g, _BNA, _BNB, dst_hbm, dst_b, isem_b)
            gb = fire_g(src_b, bufs_b, gsem_b)
            wait_g(src_a, bufs_a, gsem_a)
            stage_async(g + 1, 0, _BNA, src_hbm, src_a, isem_a)
            drain(fire_s(dst_a, bufs_a, ssem_a))
            stage_async(g + 1, 0, _BNA, dst_hbm, dst_a, isem_a)
            stage_wait(g + 1, 0, _BNA, src_hbm, src_a, isem_a)
            stage_wait(g + 1, 0, _BNA, dst_hbm, dst_a, isem_a)
            fire_g(src_a, bufs_a, gsem_a)
            drain(gb)
            drain(fire_s(dst_b, bufs_b, ssem_b))
            stage_async(g + 1, _BNA, _BNB, src_hbm, src_b, isem_b)
            stage_async(g + 1, _BNA, _BNB, dst_hbm, dst_b, isem_b)
            return carry

        lax.fori_loop(0, n_grp - 1, group, 0)
        stage_wait(n_grp - 1, _BNA, _BNB, src_hbm, src_b, isem_b)
        stage_wait(n_grp - 1, _BNA, _BNB, dst_hbm, dst_b, isem_b)
        gb = fire_g(src_b, bufs_b, gsem_b)
        wait_g(src_a, bufs_a, gsem_a)
        drain(fire_s(dst_a, bufs_a, ssem_a))
        drain(gb)
        drain(fire_s(dst_b, bufs_b, ssem_b))
        plsc.subcore_barrier()
        pltpu.sync_copy(acc_sh.at[pl.ds(sid * n_tile, n_tile)],
                        out_hbm.at[pl.ds(sid * n_tile, n_tile),
                                   pl.ds(cid * 64, 64)])

    return k(pstk, src3s, dst3s, zeros_blk)


def _bn(h, g, b):
    m = jnp.mean(h, axis=0, keepdims=True)
    c = h - m
    v = jnp.mean(c * c, axis=0, keepdims=True)
    return c / jnp.sqrt(v + 1e-5) * g[None, :] + b[None, :]


def _matT(h, w):
    return lax.dot_general(h, w, (((1,), (1,)), ((), ())),
                           precision=lax.Precision.HIGHEST,
                           preferred_element_type=jnp.float32)


def _tc_pre(x, degp, bng, bnb, wp, bp, w1, *, interpret=False):
    n = x.shape[0]

    def body(x_ref, degp_ref, bng_ref, bnb_ref, wp_ref, bp_ref, w1_ref,
             p1_ref, dinv_ref):
        deg = degp_ref[0:1, :] + degp_ref[1:2, :] + 1.0
        dinv = (1.0 / jnp.sqrt(deg)).reshape(_NACC, 1)[:n]
        h = _bn(x_ref[...], bng_ref[...], bnb_ref[...])
        h = jnp.maximum(_matT(h, wp_ref[...]) + bp_ref[...][None, :], 0.0)
        p1_ref[0:n, :] = dinv * _matT(h, w1_ref[...])
        dinv_ref[...] = dinv

    return pl.pallas_call(
        body,
        out_shape=[jax.ShapeDtypeStruct((_NACC, 128), jnp.float32),
                   jax.ShapeDtypeStruct((n, 1), jnp.float32)],
        compiler_params=pltpu.CompilerParams(vmem_limit_bytes=100 * 1024 * 1024),
        interpret=interpret,
    )(x, degp, bng, bnb, wp, bp, w1)


def _tc_mid(sp, p1, dinv, b1, bng, bnb, w2, *, interpret=False):
    n = dinv.shape[0]

    def body(sp_ref, p1_ref, dinv_ref, b1_ref, bng_ref, bnb_ref, w2_ref, p2_ref):
        dinv = dinv_ref[...]
        out1 = dinv * (sp_ref[0:n, :] + p1_ref[0:n, :]) + b1_ref[...][None, :]
        h1 = jnp.maximum(_bn(out1, bng_ref[...], bnb_ref[...]), 0.0)
        p2_ref[0:n, :] = dinv * _matT(h1, w2_ref[...])

    return pl.pallas_call(
        body,
        out_shape=jax.ShapeDtypeStruct((_NACC, 128), jnp.float32),
        compiler_params=pltpu.CompilerParams(vmem_limit_bytes=100 * 1024 * 1024),
        interpret=interpret,
    )(sp, p1, dinv, b1, bng, bnb, w2)


def _tc_post(sp, p2, dinv, b2, bng, bnb, *, interpret=False):
    n = dinv.shape[0]

    def body(sp_ref, p2_ref, dinv_ref, b2_ref, bng_ref, bnb_ref, out_ref):
        out2 = dinv_ref[...] * (sp_ref[0:n, :] + p2_ref[0:n, :]) + b2_ref[...][None, :]
        out_ref[...] = _bn(out2, bng_ref[...], bnb_ref[...])

    return pl.pallas_call(
        body,
        out_shape=jax.ShapeDtypeStruct((n, 128), jnp.float32),
        compiler_params=pltpu.CompilerParams(vmem_limit_bytes=100 * 1024 * 1024),
        interpret=interpret,
    )(sp, p2, dinv, b2, bng, bnb)


def kernel(x, edge_index, bn_in_g, bn_in_b, W_proj, b_proj, W1, b1,
           bn1_g, bn1_b, W2, b2, bn2_g, bn2_b):
    n = x.shape[0]
    e = edge_index.shape[1]
    pad = _EPAD - e
    src_p = jnp.concatenate([edge_index[0], jnp.zeros((pad,), edge_index.dtype)])
    dst_p = jnp.concatenate([edge_index[1], jnp.full((pad,), _DUMMY, edge_index.dtype)])
    src3 = src_p.reshape(_NW, _CPT, _CH)
    dst3 = dst_p.reshape(_NW, _CPT, _CH)
    src3s = src_p.reshape(_NS, _CPS, _CH)
    dst3s = dst_p.reshape(_NS, _CPS, _CH)
    ones_row = jnp.ones((_CH,), jnp.float32)
    zeros_col = jnp.zeros((_NACC // _NS,), jnp.float32)
    zeros_blk = jnp.zeros((_NACC // _NS, 64), jnp.float32)

    degp = _sc_degree(dst3, ones_row, zeros_col)
    p1, dinv = _tc_pre(x, degp, bn_in_g, bn_in_b, W_proj, b_proj, W1)
    s1p = _sc_scatter(p1, src3s, dst3s, zeros_blk)
    p2 = _tc_mid(s1p, p1, dinv, b1, bn1_g, bn1_b, W2)
    s2p = _sc_scatter(p2, src3s, dst3s, zeros_blk)
    return _tc_post(s2p, p2, dinv, b2, bn2_g, bn2_b)


# speedup vs baseline: 23.9326x; 1.0138x over previous
import functools

import jax
import jax.numpy as jnp
from jax import lax
from jax.experimental import pallas as pl
from jax.experimental.pallas import tpu as pltpu
from jax.experimental.pallas import tpu_sc as plsc

_NC, _NS, _L = 2, 16, 16
_NW = _NC * _NS
_CH = 128
_CPT = 80
_CPS = 160
_BNA, _BNB = 3, 2
_EPAD = _NW * _CPT * _CH
_NACC = 10240
_DUMMY = 10016


def _mesh():
    return plsc.VectorSubcoreMesh(core_axis_name="c", subcore_axis_name="s",
                                  num_cores=_NC, num_subcores=_NS)


def _sc_degree(dst3, ones_row, zeros_col):
    n_tile = _NACC // _NS

    @functools.partial(
        pl.kernel,
        out_type=jax.ShapeDtypeStruct((_NC, _NACC), jnp.float32),
        mesh=_mesh(),
        scratch_types=[
            pltpu.VMEM((_CPT, _CH), jnp.int32),
            pltpu.VMEM((_CH,), jnp.float32),
            pltpu.VMEM((n_tile,), jnp.float32),
            pltpu.VMEM_SHARED((_NACC,), jnp.float32),
        ],
    )
    def k(dst_hbm, ones_hbm, zeros_hbm, out_hbm, idx_v, ones_v, stage_v, acc_sh):
        cid = lax.axis_index("c")
        sid = lax.axis_index("s")
        wid = cid * _NS + sid
        pltpu.sync_copy(ones_hbm, ones_v)
        pltpu.sync_copy(zeros_hbm, stage_v)
        pltpu.sync_copy(stage_v, acc_sh.at[pl.ds(sid * n_tile, n_tile)])
        pltpu.sync_copy(dst_hbm.at[wid], idx_v)
        plsc.subcore_barrier()

        def body(c, carry):
            pltpu.sync_copy(ones_v, acc_sh.at[idx_v.at[c]], add=True)
            return carry

        lax.fori_loop(0, _CPT, body, 0)
        plsc.subcore_barrier()
        pltpu.sync_copy(acc_sh.at[pl.ds(sid * n_tile, n_tile)], stage_v)
        pltpu.sync_copy(stage_v, out_hbm.at[cid, pl.ds(sid * n_tile, n_tile)])

    return k(dst3, ones_row, zeros_col)


def _sc_scatter(pstk, src3s, dst3s, zeros_blk):
    n_tile = _NACC // _NS
    n_stride = _BNA + _BNB
    n_grp = _CPS // n_stride

    @functools.partial(
        pl.kernel,
        out_type=jax.ShapeDtypeStruct((_NACC, 128), jnp.float32),
        mesh=_mesh(),
        compiler_params=pltpu.CompilerParams(use_tc_tiling_on_sc=False),
        scratch_types=[
            [pltpu.VMEM((_BNA, _CH), jnp.int32), pltpu.VMEM((_BNA, _CH), jnp.int32),
             pltpu.VMEM((_BNB, _CH), jnp.int32), pltpu.VMEM((_BNB, _CH), jnp.int32)],
            [pltpu.VMEM((_CH, 64), jnp.float32) for _ in range(_BNA + _BNB)],
            pltpu.VMEM_SHARED((_NACC, 64), jnp.float32),
            pltpu.VMEM_SHARED((_NACC, 64), jnp.float32),
            [pltpu.SemaphoreType.DMA for _ in range(6)],
        ],
    )
    def k(p_hbm, src_hbm, dst_hbm, zeros_hbm, out_hbm,
          idx_v, bufs, acc_sh, p_sh, sems):
        cid = lax.axis_index("c")
        sid = lax.axis_index("s")
        src_a, dst_a, src_b, dst_b = idx_v
        gsem_a, ssem_a, gsem_b, ssem_b, isem_a, isem_b = sems
        bufs_a, bufs_b = bufs[:_BNA], bufs[_BNA:]
        pltpu.sync_copy(zeros_hbm, acc_sh.at[pl.ds(sid * n_tile, n_tile)])
        pltpu.sync_copy(p_hbm.at[pl.ds(sid * n_tile, n_tile),
                                 pl.ds(cid * 64, 64)],
                        p_sh.at[pl.ds(sid * n_tile, n_tile)])
        plsc.subcore_barrier()

        def stage(g, off, nb, src_v, dst_v):
            pltpu.sync_copy(src_hbm.at[sid, pl.ds(g * n_stride + off, nb)],
                            src_v)
            pltpu.sync_copy(dst_hbm.at[sid, pl.ds(g * n_stride + off, nb)],
                            dst_v)

        def stage_async(g, off, nb, hbm, dst_v, sem):
            pltpu.async_copy(hbm.at[sid, pl.ds(g * n_stride + off, nb)],
                             dst_v, sem)

        def stage_wait(g, off, nb, hbm, dst_v, sem):
            pltpu.make_async_copy(hbm.at[sid, pl.ds(g * n_stride + off, nb)],
                                  dst_v, sem).wait()

        def fire_g(src_v, bank, sem):
            return [pltpu.async_copy(p_sh.at[src_v.at[kk]], bank[kk], sem)
                    for kk in range(len(bank))]

        def fire_s(dst_v, bank, sem):
            return [pltpu.async_copy(bank[kk], acc_sh.at[dst_v.at[kk]], sem,
                                     add=True)
                    for kk in range(len(bank))]

        def wait_g(src_v, bank, sem):
            for kk in range(len(bank)):
                pltpu.make_async_copy(p_sh.at[src_v.at[kk]], bank[kk],
                                      sem).wait()

        def drain(ds):
            for d in ds:
                d.wait()

        stage(0, 0, _BNA, src_a, dst_a)
        stage_async(0, _BNA, _BNB, src_hbm, src_b, isem_b)
        stage_async(0, _BNA, _BNB, dst_hbm, dst_b, isem_b)
        fire_g(src_a, bufs_a, gsem_a)

        def group(g, carry):
            stage_wait(g, _BNA, _BNB, src_hbm, src_b, isem_b)
            stage_wait(g, _BNA, _BNB, dst_hbm, dst_b, isem_b)
            gb = fire_g(src_b, bufs_b, gsem_b)
            wait_g(src_a, bufs_a, gsem_a)
            stage_async(g + 1, 0, _BNA, src_hbm, src_a, isem_a)
            drain(fire_s(dst_a, bufs_a, ssem_a))
            stage_async(g + 1, 0, _BNA, dst_hbm, dst_a, isem_a)
            stage_wait(g + 1, 0, _BNA, src_hbm, src_a, isem_a)
            stage_wait(g + 1, 0, _BNA, dst_hbm, dst_a, isem_a)
            fire_g(src_a, bufs_a, gsem_a)
            drain(gb)
            drain(fire_s(dst_b, bufs_b, ssem_b))
            stage_async(g + 1, _BNA, _BNB, src_hbm, src_b, isem_b)
            stage_async(g + 1, _BNA, _BNB, dst_hbm, dst_b, isem_b)
            return carry

        lax.fori_loop(0, n_grp - 1, group, 0)
        stage_wait(n_grp - 1, _BNA, _BNB, src_hbm, src_b, isem_b)
        stage_wait(n_grp - 1, _BNA, _BNB, dst_hbm, dst_b, isem_b)
        gb = fire_g(src_b, bufs_b, gsem_b)
        wait_g(src_a, bufs_a, gsem_a)
        drain(fire_s(dst_a, bufs_a, ssem_a))
        drain(gb)
        drain(fire_s(dst_b, bufs_b, ssem_b))
        plsc.subcore_barrier()
        pltpu.sync_copy(acc_sh.at[pl.ds(sid * n_tile, n_tile)],
                        out_hbm.at[pl.ds(sid * n_tile, n_tile),
                                   pl.ds(cid * 64, 64)])

    return k(pstk, src3s, dst3s, zeros_blk)


def _bn(h, g, b):
    m = jnp.mean(h, axis=0, keepdims=True)
    c = h - m
    v = jnp.mean(c * c, axis=0, keepdims=True)
    return c / jnp.sqrt(v + 1e-5) * g[None, :] + b[None, :]


def _matT(h, w):
    return lax.dot_general(h, w, (((1,), (1,)), ((), ())),
                           precision=lax.Precision.HIGHEST,
                           preferred_element_type=jnp.float32)


def _tc_pre0(x, bng, bnb, wp, bp, w1, *, interpret=False):
    n = x.shape[0]

    def body(x_ref, bng_ref, bnb_ref, wp_ref, bp_ref, w1_ref, q1_ref):
        h = _bn(x_ref[...], bng_ref[...], bnb_ref[...])
        h = jnp.maximum(_matT(h, wp_ref[...]) + bp_ref[...][None, :], 0.0)
        q1_ref[...] = _matT(h, w1_ref[...])

    return pl.pallas_call(
        body,
        out_shape=jax.ShapeDtypeStruct((n, 128), jnp.float32),
        compiler_params=pltpu.CompilerParams(vmem_limit_bytes=100 * 1024 * 1024),
        interpret=interpret,
    )(x, bng, bnb, wp, bp, w1)


def _tc_pre1(q1, degp, *, interpret=False):
    n = q1.shape[0]

    def body(q1_ref, degp_ref, p1_ref, dinv_ref):
        deg = degp_ref[0:1, :] + degp_ref[1:2, :] + 1.0
        dinv = (1.0 / jnp.sqrt(deg)).reshape(_NACC, 1)[:n]
        p1_ref[0:n, :] = dinv * q1_ref[...]
        dinv_ref[...] = dinv

    return pl.pallas_call(
        body,
        out_shape=[jax.ShapeDtypeStruct((_NACC, 128), jnp.float32),
                   jax.ShapeDtypeStruct((n, 1), jnp.float32)],
        compiler_params=pltpu.CompilerParams(vmem_limit_bytes=100 * 1024 * 1024),
        interpret=interpret,
    )(q1, degp)


def _tc_mid(sp, p1, dinv, b1, bng, bnb, w2, *, interpret=False):
    n = dinv.shape[0]

    def body(sp_ref, p1_ref, dinv_ref, b1_ref, bng_ref, bnb_ref, w2_ref, p2_ref):
        dinv = dinv_ref[...]
        out1 = dinv * (sp_ref[0:n, :] + p1_ref[0:n, :]) + b1_ref[...][None, :]
        h1 = jnp.maximum(_bn(out1, bng_ref[...], bnb_ref[...]), 0.0)
        p2_ref[0:n, :] = dinv * _matT(h1, w2_ref[...])

    return pl.pallas_call(
        body,
        out_shape=jax.ShapeDtypeStruct((_NACC, 128), jnp.float32),
        compiler_params=pltpu.CompilerParams(vmem_limit_bytes=100 * 1024 * 1024),
        interpret=interpret,
    )(sp, p1, dinv, b1, bng, bnb, w2)


def _tc_post(sp, p2, dinv, b2, bng, bnb, *, interpret=False):
    n = dinv.shape[0]

    def body(sp_ref, p2_ref, dinv_ref, b2_ref, bng_ref, bnb_ref, out_ref):
        out2 = dinv_ref[...] * (sp_ref[0:n, :] + p2_ref[0:n, :]) + b2_ref[...][None, :]
        out_ref[...] = _bn(out2, bng_ref[...], bnb_ref[...])

    return pl.pallas_call(
        body,
        out_shape=jax.ShapeDtypeStruct((n, 128), jnp.float32),
        compiler_params=pltpu.CompilerParams(vmem_limit_bytes=100 * 1024 * 1024),
        interpret=interpret,
    )(sp, p2, dinv, b2, bng, bnb)


def kernel(x, edge_index, bn_in_g, bn_in_b, W_proj, b_proj, W1, b1,
           bn1_g, bn1_b, W2, b2, bn2_g, bn2_b):
    n = x.shape[0]
    e = edge_index.shape[1]
    pad = _EPAD - e
    src_p = jnp.concatenate([edge_index[0], jnp.zeros((pad,), edge_index.dtype)])
    dst_p = jnp.concatenate([edge_index[1], jnp.full((pad,), _DUMMY, edge_index.dtype)])
    src3 = src_p.reshape(_NW, _CPT, _CH)
    dst3 = dst_p.reshape(_NW, _CPT, _CH)
    src3s = src_p.reshape(_NS, _CPS, _CH)
    dst3s = dst_p.reshape(_NS, _CPS, _CH)
    ones_row = jnp.ones((_CH,), jnp.float32)
    zeros_col = jnp.zeros((_NACC // _NS,), jnp.float32)
    zeros_blk = jnp.zeros((_NACC // _NS, 64), jnp.float32)

    degp = _sc_degree(dst3, ones_row, zeros_col)
    q1 = _tc_pre0(x, bn_in_g, bn_in_b, W_proj, b_proj, W1)
    p1, dinv = _tc_pre1(q1, degp)
    s1p = _sc_scatter(p1, src3s, dst3s, zeros_blk)
    p2 = _tc_mid(s1p, p1, dinv, b1, bn1_g, bn1_b, W2)
    s2p = _sc_scatter(p2, src3s, dst3s, zeros_blk)
    return _tc_post(s2p, p2, dinv, b2, bn2_g, bn2_b)
